# baseline (speedup 1.0000x reference)
import functools

import jax
import jax.numpy as jnp
from jax import lax
from jax.experimental import pallas as pl
from jax.experimental.pallas import tpu as pltpu
from jax.experimental.pallas import tpu_sc as plsc

TM = 128
TM_LOG2 = 7
L = 16


def _gate_body(x_ref, gw_ref, e0_ref, e1_ref, w0_ref, w1_ref, *, n_experts):
    x = x_ref[...]
    logits = lax.dot_general(x, gw_ref[...], (((1,), (1,)), ((), ())),
                             preferred_element_type=jnp.float32)
    m = jnp.max(logits, axis=-1, keepdims=True)
    p = jnp.exp(logits - m)
    p = p / jnp.sum(p, axis=-1, keepdims=True)
    eio = lax.broadcasted_iota(jnp.int32, p.shape, 1)
    m1 = jnp.max(p, axis=-1, keepdims=True)
    i1 = jnp.min(jnp.where(p == m1, eio, n_experts), axis=-1, keepdims=True)
    p2 = jnp.where(eio == i1, -1.0, p)
    m2 = jnp.max(p2, axis=-1, keepdims=True)
    i2 = jnp.min(jnp.where(p2 == m2, eio, n_experts), axis=-1, keepdims=True)
    denom = m1 + m2 + 1e-20
    e0_ref[...] = i1
    e1_ref[...] = i2
    w0_ref[...] = m1 / denom
    w1_ref[...] = m2 / denom


def _gating(x, gate_weight):
    t, _ = x.shape
    n_experts = gate_weight.shape[0]
    outs = pl.pallas_call(
        functools.partial(_gate_body, n_experts=n_experts),
        out_shape=[
            jax.ShapeDtypeStruct((t, 1), jnp.int32),
            jax.ShapeDtypeStruct((t, 1), jnp.int32),
            jax.ShapeDtypeStruct((t, 1), jnp.float32),
            jax.ShapeDtypeStruct((t, 1), jnp.float32),
        ],
    )(x, gate_weight)
    e0, e1, w0, w1 = outs
    return (e0.reshape(t), e1.reshape(t), w0.reshape(t), w1.reshape(t))


def _make_router(t, n_experts, nt, ntot):
    mesh = plsc.VectorSubcoreMesh(core_axis_name="c", subcore_axis_name="s")

    @functools.partial(
        pl.kernel,
        out_type=[
            jax.ShapeDtypeStruct((ntot,), jnp.int32),
            jax.ShapeDtypeStruct((ntot,), jnp.float32),
            jax.ShapeDtypeStruct((t,), jnp.int32),
            jax.ShapeDtypeStruct((t,), jnp.int32),
            jax.ShapeDtypeStruct((nt,), jnp.int32),
        ],
        mesh=mesh,
        compiler_params=pltpu.CompilerParams(needs_layout_passes=False),
        scratch_types=[
            pltpu.VMEM((t,), jnp.int32),
            pltpu.VMEM((t,), jnp.int32),
            pltpu.VMEM((t,), jnp.float32),
            pltpu.VMEM((t,), jnp.float32),
            pltpu.VMEM((L,), jnp.int32),
            pltpu.VMEM((L,), jnp.int32),
            pltpu.VMEM((ntot,), jnp.int32),
            pltpu.VMEM((ntot,), jnp.float32),
            pltpu.VMEM((t,), jnp.int32),
            pltpu.VMEM((t,), jnp.int32),
            pltpu.VMEM((nt,), jnp.int32),
        ],
    )
    def router(e0_hbm, e1_hbm, w0_hbm, w1_hbm,
               stok_hbm, sw_hbm, inv0_hbm, inv1_hbm, te_hbm,
               e0v, e1v, w0v, w1v, offv, tmpv, stokv, swv, inv0v, inv1v, tev):
        wid = lax.axis_index("s") * 2 + lax.axis_index("c")

        @pl.when(wid == 0)
        def _():
            pltpu.sync_copy(e0_hbm, e0v)
            pltpu.sync_copy(e1_hbm, e1v)
            pltpu.sync_copy(w0_hbm, w0v)
            pltpu.sync_copy(w1_hbm, w1v)

            io = lax.iota(jnp.int32, L)
            zero16 = jnp.zeros((L,), jnp.int32)

            def place(keys, vals):
                ks, vs = plsc.sort_key_val(keys, vals)
                tmpv[...] = ks
                prev = plsc.load_gather(tmpv, [jnp.maximum(io - 1, 0)])
                nxt = plsc.load_gather(tmpv, [jnp.minimum(io + 1, L - 1)])
                is_new = (ks != prev) | (io == 0)
                first = plsc.cummax(jnp.where(is_new, io, 0))
                rank = io - first
                offk = plsc.load_gather(offv, [ks])
                dest = offk + rank
                is_last = (io == L - 1) | (ks != nxt)
                plsc.store_scatter(offv, [ks], dest + 1, mask=is_last)
                return vs, dest

            offv[...] = zero16

            def hist_body(j, _):
                toks = io + j * L
                place(e0v[pl.ds(j * L, L)], toks)
                place(e1v[pl.ds(j * L, L)], toks)
                return 0

            lax.fori_loop(0, t // L, hist_body, 0)

            c = offv[...]
            pc = ((c + (TM - 1)) >> TM_LOG2) << TM_LOG2
            po_incl = plsc.cumsum(pc)
            offv[...] = po_incl - pc
            cum_nt = po_incl >> TM_LOG2
            last_e = jnp.max(jnp.where(c > 0, io, 0))
            for ci in range(nt // L):
                tvec = io + ci * L
                acc = zero16
                for e in range(n_experts):
                    ce = jnp.max(jnp.where(io == e, cum_nt, 0))
                    acc = acc + (tvec >= ce).astype(jnp.int32)
                tev[pl.ds(ci * L, L)] = jnp.minimum(acc, last_e)

            def zinit(j, _):
                stokv[pl.ds(j * L, L)] = zero16
                swv[pl.ds(j * L, L)] = jnp.zeros((L,), jnp.float32)
                return 0

            lax.fori_loop(0, ntot // L, zinit, 0)

            def place_body(j, _):
                toks = io + j * L
                vs0, d0 = place(e0v[pl.ds(j * L, L)], toks)
                plsc.store_scatter(stokv, [d0], vs0)
                plsc.store_scatter(swv, [d0], plsc.load_gather(w0v, [vs0]))
                plsc.store_scatter(inv0v, [vs0], d0)
                vs1, d1 = place(e1v[pl.ds(j * L, L)], toks)
                plsc.store_scatter(stokv, [d1], vs1)
                plsc.store_scatter(swv, [d1], plsc.load_gather(w1v, [vs1]))
                plsc.store_scatter(inv1v, [vs1], d1)
                return 0

            lax.fori_loop(0, t // L, place_body, 0)

            pltpu.sync_copy(stokv, stok_hbm)
            pltpu.sync_copy(swv, sw_hbm)
            pltpu.sync_copy(inv0v, inv0_hbm)
            pltpu.sync_copy(inv1v, inv1_hbm)
            pltpu.sync_copy(tev, te_hbm)

    return router


def _make_gather(t, d, ntot):
    mesh = plsc.VectorSubcoreMesh(core_axis_name="c", subcore_axis_name="s")
    nw = 32
    rows_per_w = ntot // nw
    cr = 8
    nbuf = 4
    chunks = rows_per_w // cr

    @functools.partial(
        pl.kernel,
        out_type=jax.ShapeDtypeStruct((ntot, d), jnp.float32),
        mesh=mesh,
        scratch_types=(
            [pltpu.VMEM((rows_per_w,), jnp.int32)]
            + [pltpu.VMEM((cr, d), jnp.float32) for _ in range(nbuf)]
            + [pltpu.SemaphoreType.DMA for _ in range(2 * nbuf)]
        ),
    )
    def gather(x_hbm, stok_hbm, xs_hbm, idxv, *bufsem):
        bufs = bufsem[:nbuf]
        gsem = bufsem[nbuf:2 * nbuf]
        wsem = bufsem[2 * nbuf:]
        wid = lax.axis_index("s") * 2 + lax.axis_index("c")
        base = wid * rows_per_w
        pltpu.sync_copy(stok_hbm.at[pl.ds(base, rows_per_w)], idxv)

        def clamp(ci, _):
            sl = pl.ds(ci * L, L)
            idxv[sl] = jnp.clip(idxv[sl], 0, t - 1)
            return 0

        lax.fori_loop(0, rows_per_w // L, clamp, 0)

        def gth(c, buf, sem):
            return pltpu.make_async_copy(
                x_hbm.at[idxv.at[pl.ds(c * cr, cr)]], buf, sem)

        def wrt(c, buf, sem):
            return pltpu.make_async_copy(
                buf, xs_hbm.at[pl.ds(base + c * cr, cr)], sem)

        for b in range(nbuf):
            gth(b, bufs[b], gsem[b]).start()

        def body(i, _):
            for b in range(nbuf):
                c = i * nbuf + b
                gth(c, bufs[b], gsem[b]).wait()
                wrt(c, bufs[b], wsem[b]).start()
                nc = c + nbuf

                @pl.when(nc < chunks)
                def _():
                    wrt(c, bufs[b], wsem[b]).wait()
                    gth(nc, bufs[b], gsem[b]).start()
            return 0

        lax.fori_loop(0, chunks // nbuf, body, 0)
        for b in range(nbuf):
            wrt(chunks - nbuf + b, bufs[b], wsem[b]).wait()

    return gather


def _gmm_body(te_ref, xs_ref, sw_ref, wg_ref, wu_ref, wd_ref, ys_ref):
    x = xs_ref[...]
    g = jnp.dot(x, wg_ref[0], preferred_element_type=jnp.float32)
    u = jnp.dot(x, wu_ref[0], preferred_element_type=jnp.float32)
    h = (g * jax.nn.sigmoid(g)) * u * sw_ref[0]
    ys_ref[...] = jnp.dot(h, wd_ref[0], preferred_element_type=jnp.float32)


def _gmm(te, xs, swr, w_gate, w_up, w_down, nt, ntot):
    _, d, f = w_gate.shape
    grid_spec = pltpu.PrefetchScalarGridSpec(
        num_scalar_prefetch=1,
        grid=(nt,),
        in_specs=[
            pl.BlockSpec((TM, d), lambda i, te_r: (i, 0)),
            pl.BlockSpec((1, TM, 1), lambda i, te_r: (i, 0, 0)),
            pl.BlockSpec((1, d, f), lambda i, te_r: (te_r[i], 0, 0)),
            pl.BlockSpec((1, d, f), lambda i, te_r: (te_r[i], 0, 0)),
            pl.BlockSpec((1, f, d), lambda i, te_r: (te_r[i], 0, 0)),
        ],
        out_specs=pl.BlockSpec((TM, d), lambda i, te_r: (i, 0)),
    )
    return pl.pallas_call(
        _gmm_body,
        grid_spec=grid_spec,
        out_shape=jax.ShapeDtypeStruct((ntot, d), jnp.float32),
    )(te, xs, swr, w_gate, w_up, w_down)


def _make_combine(t, d, ntot):
    mesh = plsc.VectorSubcoreMesh(core_axis_name="c", subcore_axis_name="s")
    nw = 32
    rows_per_w = t // nw
    chunks = rows_per_w // L
    d_vecs = d // L

    cr = 8
    pairs = rows_per_w // (2 * cr)

    @functools.partial(
        pl.kernel,
        out_type=jax.ShapeDtypeStruct((t, d), jnp.float32),
        mesh=mesh,
        scratch_types=[
            pltpu.VMEM((rows_per_w,), jnp.int32),
            pltpu.VMEM((rows_per_w,), jnp.int32),
            pltpu.VMEM((cr, d), jnp.float32),
            pltpu.VMEM((cr, d), jnp.float32),
            pltpu.VMEM((cr, d), jnp.float32),
            pltpu.VMEM((cr, d), jnp.float32),
            pltpu.SemaphoreType.DMA,
            pltpu.SemaphoreType.DMA,
            pltpu.SemaphoreType.DMA,
            pltpu.SemaphoreType.DMA,
            pltpu.SemaphoreType.DMA,
            pltpu.SemaphoreType.DMA,
        ],
    )
    def combine(ys_hbm, inv0_hbm, inv1_hbm, y_hbm,
                idx0v, idx1v, a0, b0, a1, b1,
                ga0, gb0, ga1, gb1, wa0, wa1):
        wid = lax.axis_index("s") * 2 + lax.axis_index("c")
        base = wid * rows_per_w
        pltpu.sync_copy(inv0_hbm.at[pl.ds(base, rows_per_w)], idx0v)
        pltpu.sync_copy(inv1_hbm.at[pl.ds(base, rows_per_w)], idx1v)

        def clamp(ci, _):
            sl = pl.ds(ci * L, L)
            idx0v[sl] = jnp.clip(idx0v[sl], 0, ntot - 1)
            idx1v[sl] = jnp.clip(idx1v[sl], 0, ntot - 1)
            return 0

        lax.fori_loop(0, rows_per_w // L, clamp, 0)

        def gth(c, idxv, buf, sem):
            return pltpu.make_async_copy(
                ys_hbm.at[idxv.at[pl.ds(c * cr, cr)]], buf, sem)

        def wrt(c, buf, sem):
            return pltpu.make_async_copy(
                buf, y_hbm.at[pl.ds(base + c * cr, cr)], sem)

        def addrows(av, bv):
            def row(r, _):
                def col(j, _):
                    for u in range(8):
                        s = j * (8 * L) + u * L
                        av[r, pl.ds(s, L)] = (av[r, pl.ds(s, L)]
                                              + bv[r, pl.ds(s, L)])
                    return 0

                lax.fori_loop(0, d_vecs // 8, col, 0)
                return 0

            lax.fori_loop(0, cr, row, 0)

        def body(i, _):
            c0 = 2 * i
            c1 = 2 * i + 1

            @pl.when(i > 0)
            def _():
                wrt(c0 - 2, a0, wa0).wait()

            gth(c0, idx0v, a0, ga0).start()
            gth(c0, idx1v, b0, gb0).start()

            @pl.when(i > 0)
            def _():
                wrt(c1 - 2, a1, wa1).wait()

            gth(c1, idx0v, a1, ga1).start()
            gth(c1, idx1v, b1, gb1).start()

            gth(c0, idx0v, a0, ga0).wait()
            gth(c0, idx1v, b0, gb0).wait()
            addrows(a0, b0)
            wrt(c0, a0, wa0).start()

            gth(c1, idx0v, a1, ga1).wait()
            gth(c1, idx1v, b1, gb1).wait()
            addrows(a1, b1)
            wrt(c1, a1, wa1).start()
            return 0

        lax.fori_loop(0, pairs, body, 0)
        wrt(2 * pairs - 2, a0, wa0).wait()
        wrt(2 * pairs - 1, a1, wa1).wait()

    return combine


def kernel(hidden_states, gate_weight, w_gate, w_up, w_down):
    bsz, seq, d = hidden_states.shape
    n_experts, _, f = w_gate.shape
    x = hidden_states.reshape(-1, d)
    t = x.shape[0]
    k = 2
    nt = (t * k) // TM + n_experts
    ntot = nt * TM

    e0, e1, w0, w1 = _gating(x, gate_weight)
    stok, sw, inv0, inv1, te = _make_router(t, n_experts, nt, ntot)(
        e0, e1, w0, w1)
    xs = _make_gather(t, d, ntot)(x, stok)
    ys = _gmm(te, xs, sw.reshape(nt, TM, 1), w_gate, w_up, w_down, nt, ntot)
    y = _make_combine(t, d, ntot)(ys, inv0, inv1)
    return y.reshape(bsz, seq, d)


# speedup vs baseline: 1.1121x; 1.1121x over previous
import functools

import jax
import jax.numpy as jnp
from jax import lax
from jax.experimental import pallas as pl
from jax.experimental.pallas import tpu as pltpu
from jax.experimental.pallas import tpu_sc as plsc

TM = 128
TM_LOG2 = 7
L = 16


def _gate_body(x_ref, gw_ref, e0_ref, e1_ref, w0_ref, w1_ref, xb_ref, *,
               n_experts):
    x = x_ref[...]
    xb_ref[...] = x.astype(jnp.bfloat16)
    logits = lax.dot_general(x, gw_ref[...], (((1,), (1,)), ((), ())),
                             preferred_element_type=jnp.float32)
    m = jnp.max(logits, axis=-1, keepdims=True)
    p = jnp.exp(logits - m)
    p = p / jnp.sum(p, axis=-1, keepdims=True)
    eio = lax.broadcasted_iota(jnp.int32, p.shape, 1)
    m1 = jnp.max(p, axis=-1, keepdims=True)
    i1 = jnp.min(jnp.where(p == m1, eio, n_experts), axis=-1, keepdims=True)
    p2 = jnp.where(eio == i1, -1.0, p)
    m2 = jnp.max(p2, axis=-1, keepdims=True)
    i2 = jnp.min(jnp.where(p2 == m2, eio, n_experts), axis=-1, keepdims=True)
    denom = m1 + m2 + 1e-20
    e0_ref[...] = i1
    e1_ref[...] = i2
    w0_ref[...] = m1 / denom
    w1_ref[...] = m2 / denom


def _gating(x, gate_weight):
    t, _ = x.shape
    n_experts = gate_weight.shape[0]
    outs = pl.pallas_call(
        functools.partial(_gate_body, n_experts=n_experts),
        out_shape=[
            jax.ShapeDtypeStruct((t, 1), jnp.int32),
            jax.ShapeDtypeStruct((t, 1), jnp.int32),
            jax.ShapeDtypeStruct((t, 1), jnp.float32),
            jax.ShapeDtypeStruct((t, 1), jnp.float32),
            jax.ShapeDtypeStruct((t, gate_weight.shape[1]), jnp.bfloat16),
        ],
    )(x, gate_weight)
    e0, e1, w0, w1, xb = outs
    return (e0.reshape(t), e1.reshape(t), w0.reshape(t), w1.reshape(t), xb)


def _make_router(t, n_experts, nt, ntot):
    mesh = plsc.VectorSubcoreMesh(core_axis_name="c", subcore_axis_name="s")

    @functools.partial(
        pl.kernel,
        out_type=[
            jax.ShapeDtypeStruct((ntot,), jnp.int32),
            jax.ShapeDtypeStruct((ntot,), jnp.float32),
            jax.ShapeDtypeStruct((t,), jnp.int32),
            jax.ShapeDtypeStruct((t,), jnp.int32),
            jax.ShapeDtypeStruct((nt,), jnp.int32),
        ],
        mesh=mesh,
        compiler_params=pltpu.CompilerParams(needs_layout_passes=False),
        scratch_types=[
            pltpu.VMEM((t,), jnp.int32),
            pltpu.VMEM((t,), jnp.int32),
            pltpu.VMEM((t,), jnp.float32),
            pltpu.VMEM((t,), jnp.float32),
            pltpu.VMEM((L,), jnp.int32),
            pltpu.VMEM((L,), jnp.int32),
            pltpu.VMEM((ntot,), jnp.int32),
            pltpu.VMEM((ntot,), jnp.float32),
            pltpu.VMEM((t,), jnp.int32),
            pltpu.VMEM((t,), jnp.int32),
            pltpu.VMEM((nt,), jnp.int32),
        ],
    )
    def router(e0_hbm, e1_hbm, w0_hbm, w1_hbm,
               stok_hbm, sw_hbm, inv0_hbm, inv1_hbm, te_hbm,
               e0v, e1v, w0v, w1v, offv, tmpv, stokv, swv, inv0v, inv1v, tev):
        wid = lax.axis_index("s") * 2 + lax.axis_index("c")

        @pl.when(wid == 0)
        def _():
            pltpu.sync_copy(e0_hbm, e0v)
            pltpu.sync_copy(e1_hbm, e1v)
            pltpu.sync_copy(w0_hbm, w0v)
            pltpu.sync_copy(w1_hbm, w1v)

            io = lax.iota(jnp.int32, L)
            zero16 = jnp.zeros((L,), jnp.int32)

            def place(keys, vals):
                ks, vs = plsc.sort_key_val(keys, vals)
                tmpv[...] = ks
                prev = plsc.load_gather(tmpv, [jnp.maximum(io - 1, 0)])
                nxt = plsc.load_gather(tmpv, [jnp.minimum(io + 1, L - 1)])
                is_new = (ks != prev) | (io == 0)
                first = plsc.cummax(jnp.where(is_new, io, 0))
                rank = io - first
                offk = plsc.load_gather(offv, [ks])
                dest = offk + rank
                is_last = (io == L - 1) | (ks != nxt)
                plsc.store_scatter(offv, [ks], dest + 1, mask=is_last)
                return vs, dest

            offv[...] = zero16

            def hist_body(j, _):
                toks = io + j * L
                place(e0v[pl.ds(j * L, L)], toks)
                place(e1v[pl.ds(j * L, L)], toks)
                return 0

            lax.fori_loop(0, t // L, hist_body, 0)

            c = offv[...]
            pc = ((c + (TM - 1)) >> TM_LOG2) << TM_LOG2
            po_incl = plsc.cumsum(pc)
            offv[...] = po_incl - pc
            cum_nt = po_incl >> TM_LOG2
            last_e = jnp.max(jnp.where(c > 0, io, 0))
            for ci in range(nt // L):
                tvec = io + ci * L
                acc = zero16
                for e in range(n_experts):
                    ce = jnp.max(jnp.where(io == e, cum_nt, 0))
                    acc = acc + (tvec >= ce).astype(jnp.int32)
                tev[pl.ds(ci * L, L)] = jnp.minimum(acc, last_e)

            def zinit(j, _):
                stokv[pl.ds(j * L, L)] = zero16
                swv[pl.ds(j * L, L)] = jnp.zeros((L,), jnp.float32)
                return 0

            lax.fori_loop(0, ntot // L, zinit, 0)

            def place_body(j, _):
                toks = io + j * L
                vs0, d0 = place(e0v[pl.ds(j * L, L)], toks)
                plsc.store_scatter(stokv, [d0], vs0)
                plsc.store_scatter(swv, [d0], plsc.load_gather(w0v, [vs0]))
                plsc.store_scatter(inv0v, [vs0], d0)
                vs1, d1 = place(e1v[pl.ds(j * L, L)], toks)
                plsc.store_scatter(stokv, [d1], vs1)
                plsc.store_scatter(swv, [d1], plsc.load_gather(w1v, [vs1]))
                plsc.store_scatter(inv1v, [vs1], d1)
                return 0

            lax.fori_loop(0, t // L, place_body, 0)

            pltpu.sync_copy(stokv, stok_hbm)
            pltpu.sync_copy(swv, sw_hbm)
            pltpu.sync_copy(inv0v, inv0_hbm)
            pltpu.sync_copy(inv1v, inv1_hbm)
            pltpu.sync_copy(tev, te_hbm)

    return router


def _make_gather(t, d, ntot):
    mesh = plsc.VectorSubcoreMesh(core_axis_name="c", subcore_axis_name="s")
    nw = 32
    rows_per_w = ntot // nw
    cr = 16
    nbuf = 4
    chunks = rows_per_w // cr

    @functools.partial(
        pl.kernel,
        out_type=jax.ShapeDtypeStruct((ntot, d), jnp.int32),
        mesh=mesh,
        scratch_types=(
            [pltpu.VMEM((rows_per_w,), jnp.int32)]
            + [pltpu.VMEM((cr, d), jnp.int32) for _ in range(nbuf)]
            + [pltpu.SemaphoreType.DMA for _ in range(2 * nbuf)]
        ),
    )
    def gather(x_hbm, stok_hbm, xs_hbm, idxv, *bufsem):
        bufs = bufsem[:nbuf]
        gsem = bufsem[nbuf:2 * nbuf]
        wsem = bufsem[2 * nbuf:]
        wid = lax.axis_index("s") * 2 + lax.axis_index("c")
        base = wid * rows_per_w
        pltpu.sync_copy(stok_hbm.at[pl.ds(base, rows_per_w)], idxv)

        def clamp(ci, _):
            sl = pl.ds(ci * L, L)
            idxv[sl] = jnp.clip(idxv[sl], 0, t - 1)
            return 0

        lax.fori_loop(0, rows_per_w // L, clamp, 0)

        def gth(c, buf, sem):
            return pltpu.make_async_copy(
                x_hbm.at[idxv.at[pl.ds(c * cr, cr)]], buf, sem)

        def wrt(c, buf, sem):
            return pltpu.make_async_copy(
                buf, xs_hbm.at[pl.ds(base + c * cr, cr)], sem)

        for b in range(nbuf):
            gth(b, bufs[b], gsem[b]).start()

        def body(i, _):
            for b in range(nbuf):
                c = i * nbuf + b
                gth(c, bufs[b], gsem[b]).wait()
                wrt(c, bufs[b], wsem[b]).start()
                nc = c + nbuf

                @pl.when(nc < chunks)
                def _():
                    wrt(c, bufs[b], wsem[b]).wait()
                    gth(nc, bufs[b], gsem[b]).start()
            return 0

        lax.fori_loop(0, chunks // nbuf, body, 0)
        for b in range(nbuf):
            wrt(chunks - nbuf + b, bufs[b], wsem[b]).wait()

    return gather


def _gmm_body(te_ref, xs_ref, sw_ref, wg_ref, wu_ref, wd_ref, ys_ref):
    xi = xs_ref[...]
    x = pltpu.bitcast(xi, jnp.bfloat16)
    x = x.reshape(xi.shape[0], xi.shape[1] * 2)
    g = jnp.dot(x, wg_ref[0].astype(jnp.bfloat16),
                preferred_element_type=jnp.float32)
    u = jnp.dot(x, wu_ref[0].astype(jnp.bfloat16),
                preferred_element_type=jnp.float32)
    h = (g * jax.nn.sigmoid(g)) * u * sw_ref[0]
    ys_ref[...] = jnp.dot(h.astype(jnp.bfloat16),
                          wd_ref[0].astype(jnp.bfloat16),
                          preferred_element_type=jnp.float32)


def _gmm(te, xs, swr, w_gate, w_up, w_down, nt, ntot):
    _, d, f = w_gate.shape
    grid_spec = pltpu.PrefetchScalarGridSpec(
        num_scalar_prefetch=1,
        grid=(nt,),
        in_specs=[
            pl.BlockSpec((TM, d // 2), lambda i, te_r: (i, 0)),
            pl.BlockSpec((1, TM, 1), lambda i, te_r: (i, 0, 0)),
            pl.BlockSpec((1, d, f), lambda i, te_r: (te_r[i], 0, 0)),
            pl.BlockSpec((1, d, f), lambda i, te_r: (te_r[i], 0, 0)),
            pl.BlockSpec((1, f, d), lambda i, te_r: (te_r[i], 0, 0)),
        ],
        out_specs=pl.BlockSpec((TM, d), lambda i, te_r: (i, 0)),
    )
    return pl.pallas_call(
        _gmm_body,
        grid_spec=grid_spec,
        out_shape=jax.ShapeDtypeStruct((ntot, d), jnp.float32),
    )(te, xs, swr, w_gate, w_up, w_down)


def _make_combine(t, d, ntot):
    mesh = plsc.VectorSubcoreMesh(core_axis_name="c", subcore_axis_name="s")
    nw = 32
    rows_per_w = t // nw
    chunks = rows_per_w // L
    d_vecs = d // L

    cr = 8
    pairs = rows_per_w // (2 * cr)

    @functools.partial(
        pl.kernel,
        out_type=jax.ShapeDtypeStruct((t, d), jnp.float32),
        mesh=mesh,
        scratch_types=[
            pltpu.VMEM((rows_per_w,), jnp.int32),
            pltpu.VMEM((rows_per_w,), jnp.int32),
            pltpu.VMEM((cr, d), jnp.float32),
            pltpu.VMEM((cr, d), jnp.float32),
            pltpu.VMEM((cr, d), jnp.float32),
            pltpu.VMEM((cr, d), jnp.float32),
            pltpu.SemaphoreType.DMA,
            pltpu.SemaphoreType.DMA,
            pltpu.SemaphoreType.DMA,
            pltpu.SemaphoreType.DMA,
            pltpu.SemaphoreType.DMA,
            pltpu.SemaphoreType.DMA,
        ],
    )
    def combine(ys_hbm, inv0_hbm, inv1_hbm, y_hbm,
                idx0v, idx1v, a0, b0, a1, b1,
                ga0, gb0, ga1, gb1, wa0, wa1):
        wid = lax.axis_index("s") * 2 + lax.axis_index("c")
        base = wid * rows_per_w
        pltpu.sync_copy(inv0_hbm.at[pl.ds(base, rows_per_w)], idx0v)
        pltpu.sync_copy(inv1_hbm.at[pl.ds(base, rows_per_w)], idx1v)

        def clamp(ci, _):
            sl = pl.ds(ci * L, L)
            idx0v[sl] = jnp.clip(idx0v[sl], 0, ntot - 1)
            idx1v[sl] = jnp.clip(idx1v[sl], 0, ntot - 1)
            return 0

        lax.fori_loop(0, rows_per_w // L, clamp, 0)

        def gth(c, idxv, buf, sem):
            return pltpu.make_async_copy(
                ys_hbm.at[idxv.at[pl.ds(c * cr, cr)]], buf, sem)

        def wrt(c, buf, sem):
            return pltpu.make_async_copy(
                buf, y_hbm.at[pl.ds(base + c * cr, cr)], sem)

        def addrows(av, bv):
            def row(r, _):
                def col(j, _):
                    for u in range(8):
                        s = j * (8 * L) + u * L
                        av[r, pl.ds(s, L)] = (av[r, pl.ds(s, L)]
                                              + bv[r, pl.ds(s, L)])
                    return 0

                lax.fori_loop(0, d_vecs // 8, col, 0)
                return 0

            lax.fori_loop(0, cr, row, 0)

        def body(i, _):
            c0 = 2 * i
            c1 = 2 * i + 1

            @pl.when(i > 0)
            def _():
                wrt(c0 - 2, a0, wa0).wait()

            gth(c0, idx0v, a0, ga0).start()
            gth(c0, idx1v, b0, gb0).start()

            @pl.when(i > 0)
            def _():
                wrt(c1 - 2, a1, wa1).wait()

            gth(c1, idx0v, a1, ga1).start()
            gth(c1, idx1v, b1, gb1).start()

            gth(c0, idx0v, a0, ga0).wait()
            gth(c0, idx1v, b0, gb0).wait()
            addrows(a0, b0)
            wrt(c0, a0, wa0).start()

            gth(c1, idx0v, a1, ga1).wait()
            gth(c1, idx1v, b1, gb1).wait()
            addrows(a1, b1)
            wrt(c1, a1, wa1).start()
            return 0

        lax.fori_loop(0, pairs, body, 0)
        wrt(2 * pairs - 2, a0, wa0).wait()
        wrt(2 * pairs - 1, a1, wa1).wait()

    return combine


def kernel(hidden_states, gate_weight, w_gate, w_up, w_down):
    bsz, seq, d = hidden_states.shape
    n_experts, _, f = w_gate.shape
    x = hidden_states.reshape(-1, d)
    t = x.shape[0]
    k = 2
    nt = (t * k) // TM + n_experts
    ntot = nt * TM

    e0, e1, w0, w1, xb = _gating(x, gate_weight)
    stok, sw, inv0, inv1, te = _make_router(t, n_experts, nt, ntot)(
        e0, e1, w0, w1)
    xpair = jnp.stack([xb[:, :d // 2], xb[:, d // 2:]], axis=-1)
    xb32 = lax.bitcast_convert_type(xpair, jnp.int32)
    xs = _make_gather(t, d // 2, ntot)(xb32, stok)
    ys = _gmm(te, xs, sw.reshape(nt, TM, 1), w_gate, w_up, w_down, nt, ntot)
    y = _make_combine(t, d, ntot)(ys, inv0, inv1)
    return y.reshape(bsz, seq, d)


# speedup vs baseline: 1.1805x; 1.0614x over previous
import functools

import jax
import jax.numpy as jnp
from jax import lax
from jax.experimental import pallas as pl
from jax.experimental.pallas import tpu as pltpu
from jax.experimental.pallas import tpu_sc as plsc

TM = 128
TM_LOG2 = 7
L = 16


def _gate_body(x_ref, gw_ref, e0_ref, e1_ref, w0_ref, w1_ref, xb_ref, *,
               n_experts):
    x = x_ref[...]
    xb_ref[...] = x.astype(jnp.bfloat16)
    logits = lax.dot_general(x, gw_ref[...], (((1,), (1,)), ((), ())),
                             preferred_element_type=jnp.float32)
    m = jnp.max(logits, axis=-1, keepdims=True)
    p = jnp.exp(logits - m)
    p = p / jnp.sum(p, axis=-1, keepdims=True)
    eio = lax.broadcasted_iota(jnp.int32, p.shape, 1)
    m1 = jnp.max(p, axis=-1, keepdims=True)
    i1 = jnp.min(jnp.where(p == m1, eio, n_experts), axis=-1, keepdims=True)
    p2 = jnp.where(eio == i1, -1.0, p)
    m2 = jnp.max(p2, axis=-1, keepdims=True)
    i2 = jnp.min(jnp.where(p2 == m2, eio, n_experts), axis=-1, keepdims=True)
    denom = m1 + m2 + 1e-20
    e0_ref[...] = i1
    e1_ref[...] = i2
    w0_ref[...] = m1 / denom
    w1_ref[...] = m2 / denom


def _gating(x, gate_weight):
    t, _ = x.shape
    n_experts = gate_weight.shape[0]
    outs = pl.pallas_call(
        functools.partial(_gate_body, n_experts=n_experts),
        out_shape=[
            jax.ShapeDtypeStruct((t, 1), jnp.int32),
            jax.ShapeDtypeStruct((t, 1), jnp.int32),
            jax.ShapeDtypeStruct((t, 1), jnp.float32),
            jax.ShapeDtypeStruct((t, 1), jnp.float32),
            jax.ShapeDtypeStruct((t, gate_weight.shape[1]), jnp.bfloat16),
        ],
    )(x, gate_weight)
    e0, e1, w0, w1, xb = outs
    return (e0.reshape(t), e1.reshape(t), w0.reshape(t), w1.reshape(t), xb)


def _make_router(t, n_experts, nt, ntot):
    mesh = plsc.VectorSubcoreMesh(core_axis_name="c", subcore_axis_name="s")

    @functools.partial(
        pl.kernel,
        out_type=[
            jax.ShapeDtypeStruct((ntot,), jnp.int32),
            jax.ShapeDtypeStruct((ntot,), jnp.float32),
            jax.ShapeDtypeStruct((t,), jnp.int32),
            jax.ShapeDtypeStruct((t,), jnp.int32),
            jax.ShapeDtypeStruct((nt,), jnp.int32),
        ],
        mesh=mesh,
        compiler_params=pltpu.CompilerParams(needs_layout_passes=False),
        scratch_types=[
            pltpu.VMEM((t,), jnp.int32),
            pltpu.VMEM((t,), jnp.int32),
            pltpu.VMEM((t,), jnp.float32),
            pltpu.VMEM((t,), jnp.float32),
            pltpu.VMEM((L,), jnp.int32),
            pltpu.VMEM((L,), jnp.int32),
            pltpu.VMEM((ntot,), jnp.int32),
            pltpu.VMEM((ntot,), jnp.float32),
            pltpu.VMEM((t,), jnp.int32),
            pltpu.VMEM((t,), jnp.int32),
            pltpu.VMEM((nt,), jnp.int32),
        ],
    )
    def router(e0_hbm, e1_hbm, w0_hbm, w1_hbm,
               stok_hbm, sw_hbm, inv0_hbm, inv1_hbm, te_hbm,
               e0v, e1v, w0v, w1v, offv, tmpv, stokv, swv, inv0v, inv1v, tev):
        wid = lax.axis_index("s") * 2 + lax.axis_index("c")

        @pl.when(wid == 0)
        def _():
            pltpu.sync_copy(e0_hbm, e0v)
            pltpu.sync_copy(e1_hbm, e1v)
            pltpu.sync_copy(w0_hbm, w0v)
            pltpu.sync_copy(w1_hbm, w1v)

            io = lax.iota(jnp.int32, L)
            zero16 = jnp.zeros((L,), jnp.int32)

            def place(keys, vals):
                ks, vs = plsc.sort_key_val(keys, vals)
                tmpv[...] = ks
                prev = plsc.load_gather(tmpv, [jnp.maximum(io - 1, 0)])
                nxt = plsc.load_gather(tmpv, [jnp.minimum(io + 1, L - 1)])
                is_new = (ks != prev) | (io == 0)
                first = plsc.cummax(jnp.where(is_new, io, 0))
                rank = io - first
                offk = plsc.load_gather(offv, [ks])
                dest = offk + rank
                is_last = (io == L - 1) | (ks != nxt)
                plsc.store_scatter(offv, [ks], dest + 1, mask=is_last)
                return vs, dest

            offv[...] = zero16

            def hist_body(j, _):
                toks = io + j * L
                place(e0v[pl.ds(j * L, L)], toks)
                place(e1v[pl.ds(j * L, L)], toks)
                return 0

            lax.fori_loop(0, t // L, hist_body, 0)

            c = offv[...]
            pc = ((c + (TM - 1)) >> TM_LOG2) << TM_LOG2
            po_incl = plsc.cumsum(pc)
            offv[...] = po_incl - pc
            cum_nt = po_incl >> TM_LOG2
            last_e = jnp.max(jnp.where(c > 0, io, 0))
            for ci in range(nt // L):
                tvec = io + ci * L
                acc = zero16
                for e in range(n_experts):
                    ce = jnp.max(jnp.where(io == e, cum_nt, 0))
                    acc = acc + (tvec >= ce).astype(jnp.int32)
                tev[pl.ds(ci * L, L)] = jnp.minimum(acc, last_e)

            def zinit(j, _):
                stokv[pl.ds(j * L, L)] = zero16
                swv[pl.ds(j * L, L)] = jnp.zeros((L,), jnp.float32)
                return 0

            lax.fori_loop(0, ntot // L, zinit, 0)

            def place_body(j, _):
                toks = io + j * L
                vs0, d0 = place(e0v[pl.ds(j * L, L)], toks)
                plsc.store_scatter(stokv, [d0], vs0)
                plsc.store_scatter(swv, [d0], plsc.load_gather(w0v, [vs0]))
                plsc.store_scatter(inv0v, [vs0], d0)
                vs1, d1 = place(e1v[pl.ds(j * L, L)], toks)
                plsc.store_scatter(stokv, [d1], vs1)
                plsc.store_scatter(swv, [d1], plsc.load_gather(w1v, [vs1]))
                plsc.store_scatter(inv1v, [vs1], d1)
                return 0

            lax.fori_loop(0, t // L, place_body, 0)

            pltpu.sync_copy(stokv, stok_hbm)
            pltpu.sync_copy(swv, sw_hbm)
            pltpu.sync_copy(inv0v, inv0_hbm)
            pltpu.sync_copy(inv1v, inv1_hbm)
            pltpu.sync_copy(tev, te_hbm)

    return router


def _make_gather(t, d, ntot):
    mesh = plsc.VectorSubcoreMesh(core_axis_name="c", subcore_axis_name="s")
    nw = 32
    rows_per_w = ntot // nw
    cr = 16
    chunks = rows_per_w // cr
    nbuf = 4 if chunks % 4 == 0 else 2

    @functools.partial(
        pl.kernel,
        out_type=jax.ShapeDtypeStruct((ntot, d), jnp.int32),
        mesh=mesh,
        scratch_types=(
            [pltpu.VMEM((rows_per_w,), jnp.int32)]
            + [pltpu.VMEM((cr, d), jnp.int32) for _ in range(nbuf)]
            + [pltpu.SemaphoreType.DMA for _ in range(2 * nbuf)]
        ),
    )
    def gather(x_hbm, stok_hbm, xs_hbm, idxv, *bufsem):
        bufs = bufsem[:nbuf]
        gsem = bufsem[nbuf:2 * nbuf]
        wsem = bufsem[2 * nbuf:]
        wid = lax.axis_index("s") * 2 + lax.axis_index("c")
        base = wid * rows_per_w
        pltpu.sync_copy(stok_hbm.at[pl.ds(base, rows_per_w)], idxv)

        def clamp(ci, _):
            sl = pl.ds(ci * L, L)
            idxv[sl] = jnp.clip(idxv[sl], 0, t - 1)
            return 0

        lax.fori_loop(0, rows_per_w // L, clamp, 0)

        def gth(c, buf, sem):
            return pltpu.make_async_copy(
                x_hbm.at[idxv.at[pl.ds(c * cr, cr)]], buf, sem)

        def wrt(c, buf, sem):
            return pltpu.make_async_copy(
                buf, xs_hbm.at[pl.ds(base + c * cr, cr)], sem)

        for b in range(nbuf):
            gth(b, bufs[b], gsem[b]).start()

        def body(i, _):
            for b in range(nbuf):
                c = i * nbuf + b
                gth(c, bufs[b], gsem[b]).wait()
                wrt(c, bufs[b], wsem[b]).start()
                nc = c + nbuf

                @pl.when(nc < chunks)
                def _():
                    wrt(c, bufs[b], wsem[b]).wait()
                    gth(nc, bufs[b], gsem[b]).start()
            return 0

        lax.fori_loop(0, chunks // nbuf, body, 0)
        for b in range(nbuf):
            wrt(chunks - nbuf + b, bufs[b], wsem[b]).wait()

    return gather


def _gmm_compute(xs_ref, sw_ref, wg_ref, wu_ref, wd_ref, ys_ref):
    xi = xs_ref[...]
    x = pltpu.bitcast(xi, jnp.bfloat16)
    x = x.reshape(xi.shape[0], xi.shape[1] * 2)
    g = jnp.dot(x, wg_ref[0].astype(jnp.bfloat16),
                preferred_element_type=jnp.float32)
    u = jnp.dot(x, wu_ref[0].astype(jnp.bfloat16),
                preferred_element_type=jnp.float32)
    h = (g * jax.nn.sigmoid(g)) * u * sw_ref[0]
    ys_ref[...] = jnp.dot(h.astype(jnp.bfloat16),
                          wd_ref[0].astype(jnp.bfloat16),
                          preferred_element_type=jnp.float32)


def _gmm_body(te_ref, xs_ref, sw_ref, wg_ref, wu_ref, wd_ref, prev_ref,
              ys_ref):
    del prev_ref
    _gmm_compute(xs_ref, sw_ref, wg_ref, wu_ref, wd_ref, ys_ref)


def _gmm_body_fresh(te_ref, xs_ref, sw_ref, wg_ref, wu_ref, wd_ref, ys_ref):
    _gmm_compute(xs_ref, sw_ref, wg_ref, wu_ref, wd_ref, ys_ref)


def _gmm(te, xs, swr, w_gate, w_up, w_down, ys_prev, tile0, ngrid, ntot):
    _, d, f = w_gate.shape
    in_specs = [
        pl.BlockSpec((TM, d // 2), lambda i, te_r: (i, 0)),
        pl.BlockSpec((1, TM, 1), lambda i, te_r: (i, 0, 0)),
        pl.BlockSpec((1, d, f), lambda i, te_r: (te_r[i], 0, 0)),
        pl.BlockSpec((1, d, f), lambda i, te_r: (te_r[i], 0, 0)),
        pl.BlockSpec((1, f, d), lambda i, te_r: (te_r[i], 0, 0)),
    ]
    args = [te, xs, swr, w_gate, w_up, w_down]
    aliases = {}
    body = _gmm_body_fresh
    if ys_prev is not None:
        in_specs.append(pl.BlockSpec((TM, d), lambda i, te_r: (0, 0)))
        args.append(ys_prev)
        aliases = {6: 0}
        body = _gmm_body
    grid_spec = pltpu.PrefetchScalarGridSpec(
        num_scalar_prefetch=1,
        grid=(ngrid,),
        in_specs=in_specs,
        out_specs=pl.BlockSpec((TM, d), lambda i, te_r: (i + tile0, 0)),
    )
    return pl.pallas_call(
        body,
        grid_spec=grid_spec,
        out_shape=jax.ShapeDtypeStruct((ntot, d), jnp.float32),
        input_output_aliases=aliases,
    )(*args)


def _make_combine(t, d, ntot):
    mesh = plsc.VectorSubcoreMesh(core_axis_name="c", subcore_axis_name="s")
    nw = 32
    rows_per_w = t // nw
    chunks = rows_per_w // L
    d_vecs = d // L

    cr = 8
    pairs = rows_per_w // (2 * cr)

    @functools.partial(
        pl.kernel,
        out_type=jax.ShapeDtypeStruct((t, d), jnp.float32),
        mesh=mesh,
        scratch_types=[
            pltpu.VMEM((rows_per_w,), jnp.int32),
            pltpu.VMEM((rows_per_w,), jnp.int32),
            pltpu.VMEM((cr, d), jnp.float32),
            pltpu.VMEM((cr, d), jnp.float32),
            pltpu.VMEM((cr, d), jnp.float32),
            pltpu.VMEM((cr, d), jnp.float32),
            pltpu.SemaphoreType.DMA,
            pltpu.SemaphoreType.DMA,
            pltpu.SemaphoreType.DMA,
            pltpu.SemaphoreType.DMA,
            pltpu.SemaphoreType.DMA,
            pltpu.SemaphoreType.DMA,
        ],
    )
    def combine(ys_hbm, inv0_hbm, inv1_hbm, y_hbm,
                idx0v, idx1v, a0, b0, a1, b1,
                ga0, gb0, ga1, gb1, wa0, wa1):
        wid = lax.axis_index("s") * 2 + lax.axis_index("c")
        base = wid * rows_per_w
        pltpu.sync_copy(inv0_hbm.at[pl.ds(base, rows_per_w)], idx0v)
        pltpu.sync_copy(inv1_hbm.at[pl.ds(base, rows_per_w)], idx1v)

        def clamp(ci, _):
            sl = pl.ds(ci * L, L)
            idx0v[sl] = jnp.clip(idx0v[sl], 0, ntot - 1)
            idx1v[sl] = jnp.clip(idx1v[sl], 0, ntot - 1)
            return 0

        lax.fori_loop(0, rows_per_w // L, clamp, 0)

        def gth(c, idxv, buf, sem):
            return pltpu.make_async_copy(
                ys_hbm.at[idxv.at[pl.ds(c * cr, cr)]], buf, sem)

        def wrt(c, buf, sem):
            return pltpu.make_async_copy(
                buf, y_hbm.at[pl.ds(base + c * cr, cr)], sem)

        def addrows(av, bv):
            def row(r, _):
                def col(j, _):
                    for u in range(8):
                        s = j * (8 * L) + u * L
                        av[r, pl.ds(s, L)] = (av[r, pl.ds(s, L)]
                                              + bv[r, pl.ds(s, L)])
                    return 0

                lax.fori_loop(0, d_vecs // 8, col, 0)
                return 0

            lax.fori_loop(0, cr, row, 0)

        def body(i, _):
            c0 = 2 * i
            c1 = 2 * i + 1

            @pl.when(i > 0)
            def _():
                wrt(c0 - 2, a0, wa0).wait()

            gth(c0, idx0v, a0, ga0).start()
            gth(c0, idx1v, b0, gb0).start()

            @pl.when(i > 0)
            def _():
                wrt(c1 - 2, a1, wa1).wait()

            gth(c1, idx0v, a1, ga1).start()
            gth(c1, idx1v, b1, gb1).start()

            gth(c0, idx0v, a0, ga0).wait()
            gth(c0, idx1v, b0, gb0).wait()
            addrows(a0, b0)
            wrt(c0, a0, wa0).start()

            gth(c1, idx0v, a1, ga1).wait()
            gth(c1, idx1v, b1, gb1).wait()
            addrows(a1, b1)
            wrt(c1, a1, wa1).start()
            return 0

        lax.fori_loop(0, pairs, body, 0)
        wrt(2 * pairs - 2, a0, wa0).wait()
        wrt(2 * pairs - 1, a1, wa1).wait()

    return combine


def kernel(hidden_states, gate_weight, w_gate, w_up, w_down):
    bsz, seq, d = hidden_states.shape
    n_experts, _, f = w_gate.shape
    x = hidden_states.reshape(-1, d)
    t = x.shape[0]
    k = 2
    nt = (t * k) // TM + n_experts
    ntot = nt * TM

    e0, e1, w0, w1, xb = _gating(x, gate_weight)
    stok, sw, inv0, inv1, te = _make_router(t, n_experts, nt, ntot)(
        e0, e1, w0, w1)
    xpair = jnp.stack([xb[:, :d // 2], xb[:, d // 2:]], axis=-1)
    xb32 = lax.bitcast_convert_type(xpair, jnp.int32)

    half_nt = nt // 2
    half_rows = half_nt * TM
    swr = sw.reshape(nt, TM, 1)
    gather_half = _make_gather(t, d // 2, half_rows)
    xs_a = gather_half(xb32, lax.slice(stok, (0,), (half_rows,)))
    xs_b = gather_half(xb32, lax.slice(stok, (half_rows,), (ntot,)))
    ys_a = _gmm(te[:half_nt], xs_a, swr[:half_nt], w_gate, w_up, w_down,
                None, 0, half_nt, ntot)
    ys = _gmm(te[half_nt:], xs_b, swr[half_nt:], w_gate, w_up, w_down,
              ys_a, half_nt, half_nt, ntot)
    y = _make_combine(t, d, ntot)(ys, inv0, inv1)
    return y.reshape(bsz, seq, d)


# speedup vs baseline: 1.5407x; 1.3051x over previous
import functools

import jax
import jax.numpy as jnp
from jax import lax
from jax.experimental import pallas as pl
from jax.experimental.pallas import tpu as pltpu
from jax.experimental.pallas import tpu_sc as plsc

TM = 128
TM_LOG2 = 7
L = 16


def _gate_body(x_ref, gw_ref, e0_ref, e1_ref, w0_ref, w1_ref, xb_ref, *,
               n_experts):
    x = x_ref[...]
    xb_ref[...] = x.astype(jnp.bfloat16)
    logits = lax.dot_general(x, gw_ref[...], (((1,), (1,)), ((), ())),
                             preferred_element_type=jnp.float32)
    m = jnp.max(logits, axis=-1, keepdims=True)
    p = jnp.exp(logits - m)
    p = p / jnp.sum(p, axis=-1, keepdims=True)
    eio = lax.broadcasted_iota(jnp.int32, p.shape, 1)
    m1 = jnp.max(p, axis=-1, keepdims=True)
    i1 = jnp.min(jnp.where(p == m1, eio, n_experts), axis=-1, keepdims=True)
    p2 = jnp.where(eio == i1, -1.0, p)
    m2 = jnp.max(p2, axis=-1, keepdims=True)
    i2 = jnp.min(jnp.where(p2 == m2, eio, n_experts), axis=-1, keepdims=True)
    denom = m1 + m2 + 1e-20
    e0_ref[...] = i1
    e1_ref[...] = i2
    w0_ref[...] = m1 / denom
    w1_ref[...] = m2 / denom


def _gating(x, gate_weight):
    t, _ = x.shape
    n_experts = gate_weight.shape[0]
    outs = pl.pallas_call(
        functools.partial(_gate_body, n_experts=n_experts),
        out_shape=[
            jax.ShapeDtypeStruct((t, 1), jnp.int32),
            jax.ShapeDtypeStruct((t, 1), jnp.int32),
            jax.ShapeDtypeStruct((t, 1), jnp.float32),
            jax.ShapeDtypeStruct((t, 1), jnp.float32),
            jax.ShapeDtypeStruct((t, gate_weight.shape[1]), jnp.bfloat16),
        ],
    )(x, gate_weight)
    e0, e1, w0, w1, xb = outs
    return (e0.reshape(t), e1.reshape(t), w0.reshape(t), w1.reshape(t), xb)


def _make_router(t, n_experts, nt, ntot):
    mesh = plsc.VectorSubcoreMesh(core_axis_name="c", subcore_axis_name="s")

    @functools.partial(
        pl.kernel,
        out_type=[
            jax.ShapeDtypeStruct((ntot,), jnp.int32),
            jax.ShapeDtypeStruct((ntot,), jnp.float32),
            jax.ShapeDtypeStruct((t,), jnp.int32),
            jax.ShapeDtypeStruct((t,), jnp.int32),
            jax.ShapeDtypeStruct((nt,), jnp.int32),
        ],
        mesh=mesh,
        compiler_params=pltpu.CompilerParams(needs_layout_passes=False),
        scratch_types=[
            pltpu.VMEM((t,), jnp.int32),
            pltpu.VMEM((t,), jnp.int32),
            pltpu.VMEM((t,), jnp.float32),
            pltpu.VMEM((t,), jnp.float32),
            pltpu.VMEM((L,), jnp.int32),
            pltpu.VMEM((L,), jnp.int32),
            pltpu.VMEM((ntot,), jnp.int32),
            pltpu.VMEM((ntot,), jnp.float32),
            pltpu.VMEM((t,), jnp.int32),
            pltpu.VMEM((t,), jnp.int32),
            pltpu.VMEM((nt,), jnp.int32),
        ],
    )
    def router(e0_hbm, e1_hbm, w0_hbm, w1_hbm,
               stok_hbm, sw_hbm, inv0_hbm, inv1_hbm, te_hbm,
               e0v, e1v, w0v, w1v, offv, tmpv, stokv, swv, inv0v, inv1v, tev):
        wid = lax.axis_index("s") * 2 + lax.axis_index("c")

        @pl.when(wid == 0)
        def _():
            pltpu.sync_copy(e0_hbm, e0v)
            pltpu.sync_copy(e1_hbm, e1v)
            pltpu.sync_copy(w0_hbm, w0v)
            pltpu.sync_copy(w1_hbm, w1v)

            io = lax.iota(jnp.int32, L)
            zero16 = jnp.zeros((L,), jnp.int32)

            def place(keys, vals):
                ks, vs = plsc.sort_key_val(keys, vals)
                tmpv[...] = ks
                prev = plsc.load_gather(tmpv, [jnp.maximum(io - 1, 0)])
                nxt = plsc.load_gather(tmpv, [jnp.minimum(io + 1, L - 1)])
                is_new = (ks != prev) | (io == 0)
                first = plsc.cummax(jnp.where(is_new, io, 0))
                rank = io - first
                offk = plsc.load_gather(offv, [ks])
                dest = offk + rank
                is_last = (io == L - 1) | (ks != nxt)
                plsc.store_scatter(offv, [ks], dest + 1, mask=is_last)
                return vs, dest

            offv[...] = zero16

            def hist_body(j, _):
                toks = io + j * L
                place(e0v[pl.ds(j * L, L)], toks)
                place(e1v[pl.ds(j * L, L)], toks)
                return 0

            lax.fori_loop(0, t // L, hist_body, 0)

            c = offv[...]
            pc = ((c + (TM - 1)) >> TM_LOG2) << TM_LOG2
            po_incl = plsc.cumsum(pc)
            offv[...] = po_incl - pc
            cum_nt = po_incl >> TM_LOG2
            last_e = jnp.max(jnp.where(c > 0, io, 0))
            for ci in range(nt // L):
                tvec = io + ci * L
                acc = zero16
                for e in range(n_experts):
                    ce = jnp.max(jnp.where(io == e, cum_nt, 0))
                    acc = acc + (tvec >= ce).astype(jnp.int32)
                tev[pl.ds(ci * L, L)] = jnp.minimum(acc, last_e)

            def zinit(j, _):
                stokv[pl.ds(j * L, L)] = (io + j * L) & (t - 1)
                swv[pl.ds(j * L, L)] = jnp.zeros((L,), jnp.float32)
                return 0

            lax.fori_loop(0, ntot // L, zinit, 0)

            def place_body(j, _):
                toks = io + j * L
                vs0, d0 = place(e0v[pl.ds(j * L, L)], toks)
                plsc.store_scatter(stokv, [d0], vs0)
                plsc.store_scatter(swv, [d0], plsc.load_gather(w0v, [vs0]))
                plsc.store_scatter(inv0v, [vs0], d0)
                vs1, d1 = place(e1v[pl.ds(j * L, L)], toks)
                plsc.store_scatter(stokv, [d1], vs1)
                plsc.store_scatter(swv, [d1], plsc.load_gather(w1v, [vs1]))
                plsc.store_scatter(inv1v, [vs1], d1)
                return 0

            lax.fori_loop(0, t // L, place_body, 0)

            pltpu.sync_copy(stokv, stok_hbm)
            pltpu.sync_copy(swv, sw_hbm)
            pltpu.sync_copy(inv0v, inv0_hbm)
            pltpu.sync_copy(inv1v, inv1_hbm)
            pltpu.sync_copy(tev, te_hbm)

    return router


def _make_gather(t, d, ntot):
    mesh = plsc.VectorSubcoreMesh(core_axis_name="c", subcore_axis_name="s")
    nw = 32
    rows_per_w = ntot // nw
    cr = 16
    chunks = rows_per_w // cr
    nbuf = 4 if chunks % 4 == 0 else 2

    @functools.partial(
        pl.kernel,
        out_type=jax.ShapeDtypeStruct((ntot, d), jnp.int32),
        mesh=mesh,
        scratch_types=(
            [pltpu.VMEM((rows_per_w,), jnp.int32)]
            + [pltpu.VMEM((cr, d), jnp.int32) for _ in range(nbuf)]
            + [pltpu.SemaphoreType.DMA for _ in range(2 * nbuf)]
        ),
    )
    def gather(x_hbm, stok_hbm, xs_hbm, idxv, *bufsem):
        bufs = bufsem[:nbuf]
        gsem = bufsem[nbuf:2 * nbuf]
        wsem = bufsem[2 * nbuf:]
        wid = lax.axis_index("s") * 2 + lax.axis_index("c")
        base = wid * rows_per_w
        pltpu.sync_copy(stok_hbm.at[pl.ds(base, rows_per_w)], idxv)

        def clamp(ci, _):
            sl = pl.ds(ci * L, L)
            idxv[sl] = jnp.clip(idxv[sl], 0, t - 1)
            return 0

        lax.fori_loop(0, rows_per_w // L, clamp, 0)

        def gth(c, buf, sem):
            return pltpu.make_async_copy(
                x_hbm.at[idxv.at[pl.ds(c * cr, cr)]], buf, sem)

        def wrt(c, buf, sem):
            return pltpu.make_async_copy(
                buf, xs_hbm.at[pl.ds(base + c * cr, cr)], sem)

        for b in range(nbuf):
            gth(b, bufs[b], gsem[b]).start()

        def body(i, _):
            for b in range(nbuf):
                c = i * nbuf + b
                gth(c, bufs[b], gsem[b]).wait()
                wrt(c, bufs[b], wsem[b]).start()
                nc = c + nbuf

                @pl.when(nc < chunks)
                def _():
                    wrt(c, bufs[b], wsem[b]).wait()
                    gth(nc, bufs[b], gsem[b]).start()
            return 0

        lax.fori_loop(0, chunks // nbuf, body, 0)
        for b in range(nbuf):
            wrt(chunks - nbuf + b, bufs[b], wsem[b]).wait()

    return gather


def _gmm_compute(xs_ref, sw_ref, wg_ref, wu_ref, wd_ref, ys_ref):
    xi = xs_ref[...]
    x = pltpu.bitcast(xi, jnp.bfloat16)
    x = x.reshape(xi.shape[0], xi.shape[1] * 2)
    g = jnp.dot(x, wg_ref[0].astype(jnp.bfloat16),
                preferred_element_type=jnp.float32)
    u = jnp.dot(x, wu_ref[0].astype(jnp.bfloat16),
                preferred_element_type=jnp.float32)
    h = (g * jax.nn.sigmoid(g)) * u * sw_ref[0]
    ys_ref[...] = jnp.dot(h.astype(jnp.bfloat16),
                          wd_ref[0].astype(jnp.bfloat16),
                          preferred_element_type=jnp.float32)


def _gmm_body(te_ref, xs_ref, sw_ref, wg_ref, wu_ref, wd_ref, prev_ref,
              ys_ref):
    del prev_ref
    _gmm_compute(xs_ref, sw_ref, wg_ref, wu_ref, wd_ref, ys_ref)


def _gmm_body_fresh(te_ref, xs_ref, sw_ref, wg_ref, wu_ref, wd_ref, ys_ref):
    _gmm_compute(xs_ref, sw_ref, wg_ref, wu_ref, wd_ref, ys_ref)


def _gmm(te, xs, swr, w_gate, w_up, w_down, ys_prev, tile0, ngrid, ntot):
    _, d, f = w_gate.shape
    in_specs = [
        pl.BlockSpec((TM, d // 2), lambda i, te_r: (i, 0)),
        pl.BlockSpec((1, TM, 1), lambda i, te_r: (i, 0, 0)),
        pl.BlockSpec((1, d, f), lambda i, te_r: (te_r[i], 0, 0)),
        pl.BlockSpec((1, d, f), lambda i, te_r: (te_r[i], 0, 0)),
        pl.BlockSpec((1, f, d), lambda i, te_r: (te_r[i], 0, 0)),
    ]
    args = [te, xs, swr, w_gate, w_up, w_down]
    aliases = {}
    body = _gmm_body_fresh
    if ys_prev is not None:
        in_specs.append(pl.BlockSpec((TM, d), lambda i, te_r: (0, 0)))
        args.append(ys_prev)
        aliases = {6: 0}
        body = _gmm_body
    grid_spec = pltpu.PrefetchScalarGridSpec(
        num_scalar_prefetch=1,
        grid=(ngrid,),
        in_specs=in_specs,
        out_specs=pl.BlockSpec((TM, d), lambda i, te_r: (i + tile0, 0)),
    )
    return pl.pallas_call(
        body,
        grid_spec=grid_spec,
        out_shape=jax.ShapeDtypeStruct((ntot, d), jnp.float32),
        input_output_aliases=aliases,
    )(*args)


def _make_combine(t, d, ntot):
    mesh = plsc.VectorSubcoreMesh(core_axis_name="c", subcore_axis_name="s")
    nw = 32
    rows_per_w = t // nw
    chunks = rows_per_w // L
    d_vecs = d // L

    cr = 8
    pairs = rows_per_w // (2 * cr)

    @functools.partial(
        pl.kernel,
        out_type=jax.ShapeDtypeStruct((t, d), jnp.float32),
        mesh=mesh,
        scratch_types=[
            pltpu.VMEM((rows_per_w,), jnp.int32),
            pltpu.VMEM((rows_per_w,), jnp.int32),
            pltpu.VMEM((cr, d), jnp.float32),
            pltpu.VMEM((cr, d), jnp.float32),
            pltpu.VMEM((cr, d), jnp.float32),
            pltpu.VMEM((cr, d), jnp.float32),
            pltpu.SemaphoreType.DMA,
            pltpu.SemaphoreType.DMA,
            pltpu.SemaphoreType.DMA,
            pltpu.SemaphoreType.DMA,
            pltpu.SemaphoreType.DMA,
            pltpu.SemaphoreType.DMA,
        ],
    )
    def combine(ys_hbm, inv0_hbm, inv1_hbm, y_hbm,
                idx0v, idx1v, a0, b0, a1, b1,
                ga0, gb0, ga1, gb1, wa0, wa1):
        wid = lax.axis_index("s") * 2 + lax.axis_index("c")
        base = wid * rows_per_w
        pltpu.sync_copy(inv0_hbm.at[pl.ds(base, rows_per_w)], idx0v)
        pltpu.sync_copy(inv1_hbm.at[pl.ds(base, rows_per_w)], idx1v)

        def clamp(ci, _):
            sl = pl.ds(ci * L, L)
            idx0v[sl] = jnp.clip(idx0v[sl], 0, ntot - 1)
            idx1v[sl] = jnp.clip(idx1v[sl], 0, ntot - 1)
            return 0

        lax.fori_loop(0, rows_per_w // L, clamp, 0)

        def gth(c, idxv, buf, sem):
            return pltpu.make_async_copy(
                ys_hbm.at[idxv.at[pl.ds(c * cr, cr)]], buf, sem)

        def wrt(c, buf, sem):
            return pltpu.make_async_copy(
                buf, y_hbm.at[pl.ds(base + c * cr, cr)], sem)

        def addrows(av, bv):
            def row(r, _):
                def col(j, _):
                    for u in range(8):
                        s = j * (8 * L) + u * L
                        av[r, pl.ds(s, L)] = (av[r, pl.ds(s, L)]
                                              + bv[r, pl.ds(s, L)])
                    return 0

                lax.fori_loop(0, d_vecs // 8, col, 0)
                return 0

            lax.fori_loop(0, cr, row, 0)

        def body(i, _):
            c0 = 2 * i
            c1 = 2 * i + 1

            @pl.when(i > 0)
            def _():
                wrt(c0 - 2, a0, wa0).wait()

            gth(c0, idx0v, a0, ga0).start()
            gth(c0, idx1v, b0, gb0).start()

            @pl.when(i > 0)
            def _():
                wrt(c1 - 2, a1, wa1).wait()

            gth(c1, idx0v, a1, ga1).start()
            gth(c1, idx1v, b1, gb1).start()

            gth(c0, idx0v, a0, ga0).wait()
            gth(c0, idx1v, b0, gb0).wait()
            addrows(a0, b0)
            wrt(c0, a0, wa0).start()

            gth(c1, idx0v, a1, ga1).wait()
            gth(c1, idx1v, b1, gb1).wait()
            addrows(a1, b1)
            wrt(c1, a1, wa1).start()
            return 0

        lax.fori_loop(0, pairs, body, 0)
        wrt(2 * pairs - 2, a0, wa0).wait()
        wrt(2 * pairs - 1, a1, wa1).wait()

    return combine


def kernel(hidden_states, gate_weight, w_gate, w_up, w_down):
    bsz, seq, d = hidden_states.shape
    n_experts, _, f = w_gate.shape
    x = hidden_states.reshape(-1, d)
    t = x.shape[0]
    k = 2
    nt = (t * k) // TM + n_experts
    ntot = nt * TM

    e0, e1, w0, w1, xb = _gating(x, gate_weight)
    stok, sw, inv0, inv1, te = _make_router(t, n_experts, nt, ntot)(
        e0, e1, w0, w1)
    xpair = jnp.stack([xb[:, :d // 2], xb[:, d // 2:]], axis=-1)
    xb32 = lax.bitcast_convert_type(xpair, jnp.int32)

    half_nt = nt // 2
    half_rows = half_nt * TM
    swr = sw.reshape(nt, TM, 1)
    gather_half = _make_gather(t, d // 2, half_rows)
    xs_a = gather_half(xb32, lax.slice(stok, (0,), (half_rows,)))
    xs_b = gather_half(xb32, lax.slice(stok, (half_rows,), (ntot,)))
    ys_a = _gmm(te[:half_nt], xs_a, swr[:half_nt], w_gate, w_up, w_down,
                None, 0, half_nt, ntot)
    ys = _gmm(te[half_nt:], xs_b, swr[half_nt:], w_gate, w_up, w_down,
              ys_a, half_nt, half_nt, ntot)
    y = _make_combine(t, d, ntot)(ys, inv0, inv1)
    return y.reshape(bsz, seq, d)


# speedup vs baseline: 1.6871x; 1.0951x over previous
import functools

import jax
import jax.numpy as jnp
from jax import lax
from jax.experimental import pallas as pl
from jax.experimental.pallas import tpu as pltpu
from jax.experimental.pallas import tpu_sc as plsc

TM = 128
TM_LOG2 = 7
L = 16


def _gate_body(x_ref, gw_ref, e0_ref, e1_ref, w0_ref, w1_ref, xb_ref, *,
               n_experts):
    x = x_ref[...]
    t = x.shape[0]
    xbf = x.astype(jnp.bfloat16).reshape(2 * t, x.shape[1] // 2)
    xb_ref[...] = pltpu.bitcast(xbf, jnp.int32)
    logits = lax.dot_general(x, gw_ref[...], (((1,), (1,)), ((), ())),
                             preferred_element_type=jnp.float32)
    m = jnp.max(logits, axis=-1, keepdims=True)
    p = jnp.exp(logits - m)
    p = p / jnp.sum(p, axis=-1, keepdims=True)
    eio = lax.broadcasted_iota(jnp.int32, p.shape, 1)
    m1 = jnp.max(p, axis=-1, keepdims=True)
    i1 = jnp.min(jnp.where(p == m1, eio, n_experts), axis=-1, keepdims=True)
    p2 = jnp.where(eio == i1, -1.0, p)
    m2 = jnp.max(p2, axis=-1, keepdims=True)
    i2 = jnp.min(jnp.where(p2 == m2, eio, n_experts), axis=-1, keepdims=True)
    denom = m1 + m2 + 1e-20
    e0_ref[...] = i1
    e1_ref[...] = i2
    w0_ref[...] = m1 / denom
    w1_ref[...] = m2 / denom


def _gating(x, gate_weight):
    t, _ = x.shape
    n_experts = gate_weight.shape[0]
    outs = pl.pallas_call(
        functools.partial(_gate_body, n_experts=n_experts),
        out_shape=[
            jax.ShapeDtypeStruct((t, 1), jnp.int32),
            jax.ShapeDtypeStruct((t, 1), jnp.int32),
            jax.ShapeDtypeStruct((t, 1), jnp.float32),
            jax.ShapeDtypeStruct((t, 1), jnp.float32),
            jax.ShapeDtypeStruct((t, gate_weight.shape[1] // 2), jnp.int32),
        ],
    )(x, gate_weight)
    e0, e1, w0, w1, xb = outs
    return (e0.reshape(t), e1.reshape(t), w0.reshape(t), w1.reshape(t), xb)


def _make_router(t, n_experts, nt, ntot):
    mesh = plsc.VectorSubcoreMesh(core_axis_name="c", subcore_axis_name="s")

    @functools.partial(
        pl.kernel,
        out_type=[
            jax.ShapeDtypeStruct((ntot,), jnp.int32),
            jax.ShapeDtypeStruct((ntot,), jnp.float32),
            jax.ShapeDtypeStruct((t,), jnp.int32),
            jax.ShapeDtypeStruct((t,), jnp.int32),
            jax.ShapeDtypeStruct((nt,), jnp.int32),
        ],
        mesh=mesh,
        compiler_params=pltpu.CompilerParams(needs_layout_passes=False),
        scratch_types=[
            pltpu.VMEM((t,), jnp.int32),
            pltpu.VMEM((t,), jnp.int32),
            pltpu.VMEM((t,), jnp.float32),
            pltpu.VMEM((t,), jnp.float32),
            pltpu.VMEM((L,), jnp.int32),
            pltpu.VMEM((L,), jnp.int32),
            pltpu.VMEM((ntot,), jnp.int32),
            pltpu.VMEM((ntot,), jnp.float32),
            pltpu.VMEM((t,), jnp.int32),
            pltpu.VMEM((t,), jnp.int32),
            pltpu.VMEM((nt,), jnp.int32),
        ],
    )
    def router(e0_hbm, e1_hbm, w0_hbm, w1_hbm,
               stok_hbm, sw_hbm, inv0_hbm, inv1_hbm, te_hbm,
               e0v, e1v, w0v, w1v, offv, tmpv, stokv, swv, inv0v, inv1v, tev):
        wid = lax.axis_index("s") * 2 + lax.axis_index("c")

        @pl.when(wid == 0)
        def _():
            pltpu.sync_copy(e0_hbm, e0v)
            pltpu.sync_copy(e1_hbm, e1v)
            pltpu.sync_copy(w0_hbm, w0v)
            pltpu.sync_copy(w1_hbm, w1v)

            io = lax.iota(jnp.int32, L)
            zero16 = jnp.zeros((L,), jnp.int32)

            def place(keys, vals):
                ks, vs = plsc.sort_key_val(keys, vals)
                tmpv[...] = ks
                prev = plsc.load_gather(tmpv, [jnp.maximum(io - 1, 0)])
                nxt = plsc.load_gather(tmpv, [jnp.minimum(io + 1, L - 1)])
                is_new = (ks != prev) | (io == 0)
                first = plsc.cummax(jnp.where(is_new, io, 0))
                rank = io - first
                offk = plsc.load_gather(offv, [ks])
                dest = offk + rank
                is_last = (io == L - 1) | (ks != nxt)
                plsc.store_scatter(offv, [ks], dest + 1, mask=is_last)
                return vs, dest

            offv[...] = zero16

            def hist_body(j, _):
                toks = io + j * L
                place(e0v[pl.ds(j * L, L)], toks)
                place(e1v[pl.ds(j * L, L)], toks)
                return 0

            lax.fori_loop(0, t // L, hist_body, 0)

            c = offv[...]
            pc = ((c + (TM - 1)) >> TM_LOG2) << TM_LOG2
            po_incl = plsc.cumsum(pc)
            offv[...] = po_incl - pc
            cum_nt = po_incl >> TM_LOG2
            last_e = jnp.max(jnp.where(c > 0, io, 0))
            for ci in range(nt // L):
                tvec = io + ci * L
                acc = zero16
                for e in range(n_experts):
                    ce = jnp.max(jnp.where(io == e, cum_nt, 0))
                    acc = acc + (tvec >= ce).astype(jnp.int32)
                tev[pl.ds(ci * L, L)] = jnp.minimum(acc, last_e)

            def zinit(j, _):
                stokv[pl.ds(j * L, L)] = (io + j * L) & (t - 1)
                swv[pl.ds(j * L, L)] = jnp.zeros((L,), jnp.float32)
                return 0

            lax.fori_loop(0, ntot // L, zinit, 0)

            def place_body(j, _):
                toks = io + j * L
                vs0, d0 = place(e0v[pl.ds(j * L, L)], toks)
                plsc.store_scatter(stokv, [d0], vs0)
                plsc.store_scatter(swv, [d0], plsc.load_gather(w0v, [vs0]))
                plsc.store_scatter(inv0v, [vs0], d0)
                vs1, d1 = place(e1v[pl.ds(j * L, L)], toks)
                plsc.store_scatter(stokv, [d1], vs1)
                plsc.store_scatter(swv, [d1], plsc.load_gather(w1v, [vs1]))
                plsc.store_scatter(inv1v, [vs1], d1)
                return 0

            lax.fori_loop(0, t // L, place_body, 0)

            pltpu.sync_copy(stokv, stok_hbm)
            pltpu.sync_copy(swv, sw_hbm)
            pltpu.sync_copy(inv0v, inv0_hbm)
            pltpu.sync_copy(inv1v, inv1_hbm)
            pltpu.sync_copy(tev, te_hbm)

    return router


def _make_gather(t, d, ntot):
    mesh = plsc.VectorSubcoreMesh(core_axis_name="c", subcore_axis_name="s")
    nw = 32
    rows_per_w = ntot // nw
    cr = 16
    chunks = rows_per_w // cr
    nbuf = 4 if chunks % 4 == 0 else 2

    @functools.partial(
        pl.kernel,
        out_type=jax.ShapeDtypeStruct((ntot, d), jnp.int32),
        mesh=mesh,
        scratch_types=(
            [pltpu.VMEM((rows_per_w,), jnp.int32)]
            + [pltpu.VMEM((cr, d), jnp.int32) for _ in range(nbuf)]
            + [pltpu.SemaphoreType.DMA for _ in range(2 * nbuf)]
        ),
    )
    def gather(x_hbm, stok_hbm, xs_hbm, idxv, *bufsem):
        bufs = bufsem[:nbuf]
        gsem = bufsem[nbuf:2 * nbuf]
        wsem = bufsem[2 * nbuf:]
        wid = lax.axis_index("s") * 2 + lax.axis_index("c")
        base = wid * rows_per_w
        pltpu.sync_copy(stok_hbm.at[pl.ds(base, rows_per_w)], idxv)

        def clamp(ci, _):
            sl = pl.ds(ci * L, L)
            idxv[sl] = jnp.clip(idxv[sl], 0, t - 1)
            return 0

        lax.fori_loop(0, rows_per_w // L, clamp, 0)

        def gth(c, buf, sem):
            return pltpu.make_async_copy(
                x_hbm.at[idxv.at[pl.ds(c * cr, cr)]], buf, sem)

        def wrt(c, buf, sem):
            return pltpu.make_async_copy(
                buf, xs_hbm.at[pl.ds(base + c * cr, cr)], sem)

        for b in range(nbuf):
            gth(b, bufs[b], gsem[b]).start()

        def body(i, _):
            for b in range(nbuf):
                c = i * nbuf + b
                gth(c, bufs[b], gsem[b]).wait()
                wrt(c, bufs[b], wsem[b]).start()
                nc = c + nbuf

                @pl.when(nc < chunks)
                def _():
                    wrt(c, bufs[b], wsem[b]).wait()
                    gth(nc, bufs[b], gsem[b]).start()
            return 0

        lax.fori_loop(0, chunks // nbuf, body, 0)
        for b in range(nbuf):
            wrt(chunks - nbuf + b, bufs[b], wsem[b]).wait()

    return gather


def _gmm_compute(xs_ref, sw_ref, wg_ref, wu_ref, wd_ref, ys_ref):
    xi = xs_ref[...]
    x = pltpu.bitcast(xi, jnp.bfloat16)
    x = x.reshape(xi.shape[0], xi.shape[1] * 2)
    g = jnp.dot(x, wg_ref[0].astype(jnp.bfloat16),
                preferred_element_type=jnp.float32)
    u = jnp.dot(x, wu_ref[0].astype(jnp.bfloat16),
                preferred_element_type=jnp.float32)
    h = (g * jax.nn.sigmoid(g)) * u * sw_ref[0]
    y = jnp.dot(h.astype(jnp.bfloat16), wd_ref[0].astype(jnp.bfloat16),
                preferred_element_type=jnp.float32)
    dw = y.shape[1] // 2
    yi = lax.bitcast_convert_type(y, jnp.int32)
    lo = yi[:, :dw]
    hi = yi[:, dw:]
    lob = ((lo + 0x7FFF + ((lo >> 16) & 1)) >> 16) & 0xFFFF
    hib = ((hi + 0x7FFF + ((hi >> 16) & 1)) >> 16) << 16
    ys_ref[...] = hib | lob


def _gmm_body(te_ref, xs_ref, sw_ref, wg_ref, wu_ref, wd_ref, prev_ref,
              ys_ref):
    del prev_ref
    _gmm_compute(xs_ref, sw_ref, wg_ref, wu_ref, wd_ref, ys_ref)


def _gmm_body_fresh(te_ref, xs_ref, sw_ref, wg_ref, wu_ref, wd_ref, ys_ref):
    _gmm_compute(xs_ref, sw_ref, wg_ref, wu_ref, wd_ref, ys_ref)


def _gmm(te, xs, swr, w_gate, w_up, w_down, ys_prev, tile0, ngrid, ntot):
    _, d, f = w_gate.shape
    in_specs = [
        pl.BlockSpec((TM, d // 2), lambda i, te_r: (i, 0)),
        pl.BlockSpec((1, TM, 1), lambda i, te_r: (i, 0, 0)),
        pl.BlockSpec((1, d, f), lambda i, te_r: (te_r[i], 0, 0)),
        pl.BlockSpec((1, d, f), lambda i, te_r: (te_r[i], 0, 0)),
        pl.BlockSpec((1, f, d), lambda i, te_r: (te_r[i], 0, 0)),
    ]
    args = [te, xs, swr, w_gate, w_up, w_down]
    aliases = {}
    body = _gmm_body_fresh
    if ys_prev is not None:
        in_specs.append(pl.BlockSpec((TM, d // 2), lambda i, te_r: (0, 0)))
        args.append(ys_prev)
        aliases = {6: 0}
        body = _gmm_body
    grid_spec = pltpu.PrefetchScalarGridSpec(
        num_scalar_prefetch=1,
        grid=(ngrid,),
        in_specs=in_specs,
        out_specs=pl.BlockSpec((TM, d // 2), lambda i, te_r: (i + tile0, 0)),
    )
    return pl.pallas_call(
        body,
        grid_spec=grid_spec,
        out_shape=jax.ShapeDtypeStruct((ntot, d // 2), jnp.int32),
        input_output_aliases=aliases,
    )(*args)


def _make_combine(t, d, ntot):
    mesh = plsc.VectorSubcoreMesh(core_axis_name="c", subcore_axis_name="s")
    nw = 32
    rows_per_w = t // nw
    chunks = rows_per_w // L
    d_vecs = d // L

    cr = 8
    pairs = rows_per_w // (2 * cr)
    dw = d // 2

    @functools.partial(
        pl.kernel,
        out_type=jax.ShapeDtypeStruct((t, d), jnp.float32),
        mesh=mesh,
        compiler_params=pltpu.CompilerParams(needs_layout_passes=False),
        scratch_types=[
            pltpu.VMEM((rows_per_w,), jnp.int32),
            pltpu.VMEM((rows_per_w,), jnp.int32),
            pltpu.VMEM((cr, dw), jnp.int32),
            pltpu.VMEM((cr, dw), jnp.int32),
            pltpu.VMEM((cr, dw), jnp.int32),
            pltpu.VMEM((cr, dw), jnp.int32),
            pltpu.VMEM((cr, d), jnp.float32),
            pltpu.VMEM((cr, d), jnp.float32),
            pltpu.SemaphoreType.DMA,
            pltpu.SemaphoreType.DMA,
            pltpu.SemaphoreType.DMA,
            pltpu.SemaphoreType.DMA,
            pltpu.SemaphoreType.DMA,
            pltpu.SemaphoreType.DMA,
        ],
    )
    def combine(ys_hbm, inv0_hbm, inv1_hbm, y_hbm,
                idx0v, idx1v, a0, b0, a1, b1, o0, o1,
                ga0, gb0, ga1, gb1, wa0, wa1):
        wid = lax.axis_index("s") * 2 + lax.axis_index("c")
        base = wid * rows_per_w
        pltpu.sync_copy(inv0_hbm.at[pl.ds(base, rows_per_w)], idx0v)
        pltpu.sync_copy(inv1_hbm.at[pl.ds(base, rows_per_w)], idx1v)

        def clamp(ci, _):
            sl = pl.ds(ci * L, L)
            idx0v[sl] = jnp.clip(idx0v[sl], 0, ntot - 1)
            idx1v[sl] = jnp.clip(idx1v[sl], 0, ntot - 1)
            return 0

        lax.fori_loop(0, rows_per_w // L, clamp, 0)

        def gth(c, idxv, buf, sem):
            return pltpu.make_async_copy(
                ys_hbm.at[idxv.at[pl.ds(c * cr, cr)]], buf, sem)

        def wrt(c, buf, sem):
            return pltpu.make_async_copy(
                buf, y_hbm.at[pl.ds(base + c * cr, cr)], sem)

        himask = jnp.full((L,), -65536, jnp.int32)

        def addrows(av, bv, ov):
            def row(r, _):
                def col(j, _):
                    for u in range(4):
                        s = j * (4 * L) + u * L
                        aw = av[r, pl.ds(s, L)]
                        bw = bv[r, pl.ds(s, L)]
                        sm = plsc.bitcast(
                            plsc.bitcast(aw, jnp.bfloat16)
                            + plsc.bitcast(bw, jnp.bfloat16), jnp.int32)
                        ov[r, pl.ds(s, L)] = plsc.bitcast(
                            sm << 16, jnp.float32)
                        ov[r, pl.ds(dw + s, L)] = plsc.bitcast(
                            sm & himask, jnp.float32)
                    return 0

                lax.fori_loop(0, dw // (4 * L), col, 0)
                return 0

            lax.fori_loop(0, cr, row, 0)

        def body(i, _):
            c0 = 2 * i
            c1 = 2 * i + 1

            @pl.when(i > 0)
            def _():
                wrt(c0 - 2, o0, wa0).wait()

            gth(c0, idx0v, a0, ga0).start()
            gth(c0, idx1v, b0, gb0).start()

            @pl.when(i > 0)
            def _():
                wrt(c1 - 2, o1, wa1).wait()

            gth(c1, idx0v, a1, ga1).start()
            gth(c1, idx1v, b1, gb1).start()

            gth(c0, idx0v, a0, ga0).wait()
            gth(c0, idx1v, b0, gb0).wait()
            addrows(a0, b0, o0)
            wrt(c0, o0, wa0).start()

            gth(c1, idx0v, a1, ga1).wait()
            gth(c1, idx1v, b1, gb1).wait()
            addrows(a1, b1, o1)
            wrt(c1, o1, wa1).start()
            return 0

        lax.fori_loop(0, pairs, body, 0)
        wrt(2 * pairs - 2, o0, wa0).wait()
        wrt(2 * pairs - 1, o1, wa1).wait()

    return combine


def kernel(hidden_states, gate_weight, w_gate, w_up, w_down):
    bsz, seq, d = hidden_states.shape
    n_experts, _, f = w_gate.shape
    x = hidden_states.reshape(-1, d)
    t = x.shape[0]
    k = 2
    nt = (t * k) // TM + n_experts
    ntot = nt * TM

    e0, e1, w0, w1, xb = _gating(x, gate_weight)
    stok, sw, inv0, inv1, te = _make_router(t, n_experts, nt, ntot)(
        e0, e1, w0, w1)
    xb32 = xb

    half_nt = nt // 2
    half_rows = half_nt * TM
    swr = sw.reshape(nt, TM, 1)
    gather_half = _make_gather(t, d // 2, half_rows)
    xs_a = gather_half(xb32, lax.slice(stok, (0,), (half_rows,)))
    xs_b = gather_half(xb32, lax.slice(stok, (half_rows,), (ntot,)))
    ys_a = _gmm(te[:half_nt], xs_a, swr[:half_nt], w_gate, w_up, w_down,
                None, 0, half_nt, ntot)
    ys = _gmm(te[half_nt:], xs_b, swr[half_nt:], w_gate, w_up, w_down,
              ys_a, half_nt, half_nt, ntot)
    y = _make_combine(t, d, ntot)(ys, inv0, inv1)
    return y.reshape(bsz, seq, d)


# speedup vs baseline: 1.6947x; 1.0045x over previous
import functools

import jax
import jax.numpy as jnp
from jax import lax
from jax.experimental import pallas as pl
from jax.experimental.pallas import tpu as pltpu
from jax.experimental.pallas import tpu_sc as plsc

TM = 128
TM_LOG2 = 7
L = 16


def _gate_body(x_ref, gw_ref, e0_ref, e1_ref, w0_ref, w1_ref, xb_ref, *,
               n_experts):
    x = x_ref[...]
    t = x.shape[0]
    xbf = x.astype(jnp.bfloat16).reshape(2 * t, x.shape[1] // 2)
    xb_ref[...] = pltpu.bitcast(xbf, jnp.int32)
    logits = lax.dot_general(x, gw_ref[...], (((1,), (1,)), ((), ())),
                             preferred_element_type=jnp.float32)
    m = jnp.max(logits, axis=-1, keepdims=True)
    p = jnp.exp(logits - m)
    p = p / jnp.sum(p, axis=-1, keepdims=True)
    eio = lax.broadcasted_iota(jnp.int32, p.shape, 1)
    m1 = jnp.max(p, axis=-1, keepdims=True)
    i1 = jnp.min(jnp.where(p == m1, eio, n_experts), axis=-1, keepdims=True)
    p2 = jnp.where(eio == i1, -1.0, p)
    m2 = jnp.max(p2, axis=-1, keepdims=True)
    i2 = jnp.min(jnp.where(p2 == m2, eio, n_experts), axis=-1, keepdims=True)
    denom = m1 + m2 + 1e-20
    e0_ref[...] = i1
    e1_ref[...] = i2
    w0_ref[...] = m1 / denom
    w1_ref[...] = m2 / denom


def _gating(x, gate_weight):
    t, _ = x.shape
    n_experts = gate_weight.shape[0]
    outs = pl.pallas_call(
        functools.partial(_gate_body, n_experts=n_experts),
        out_shape=[
            jax.ShapeDtypeStruct((t, 1), jnp.int32),
            jax.ShapeDtypeStruct((t, 1), jnp.int32),
            jax.ShapeDtypeStruct((t, 1), jnp.float32),
            jax.ShapeDtypeStruct((t, 1), jnp.float32),
            jax.ShapeDtypeStruct((t, gate_weight.shape[1] // 2), jnp.int32),
        ],
    )(x, gate_weight)
    e0, e1, w0, w1, xb = outs
    return (e0.reshape(t), e1.reshape(t), w0.reshape(t), w1.reshape(t), xb)


def _make_router(t, n_experts, nt, ntot):
    mesh = plsc.VectorSubcoreMesh(core_axis_name="c", subcore_axis_name="s")

    @functools.partial(
        pl.kernel,
        out_type=[
            jax.ShapeDtypeStruct((ntot,), jnp.int32),
            jax.ShapeDtypeStruct((ntot,), jnp.float32),
            jax.ShapeDtypeStruct((t,), jnp.int32),
            jax.ShapeDtypeStruct((t,), jnp.int32),
            jax.ShapeDtypeStruct((nt,), jnp.int32),
        ],
        mesh=mesh,
        compiler_params=pltpu.CompilerParams(needs_layout_passes=False),
        scratch_types=[
            pltpu.VMEM((t,), jnp.int32),
            pltpu.VMEM((t,), jnp.int32),
            pltpu.VMEM((t,), jnp.float32),
            pltpu.VMEM((t,), jnp.float32),
            pltpu.VMEM((L,), jnp.int32),
            pltpu.VMEM((L,), jnp.int32),
            pltpu.VMEM((ntot,), jnp.int32),
            pltpu.VMEM((ntot,), jnp.float32),
            pltpu.VMEM((t,), jnp.int32),
            pltpu.VMEM((t,), jnp.int32),
            pltpu.VMEM((nt,), jnp.int32),
        ],
    )
    def router(e0_hbm, e1_hbm, w0_hbm, w1_hbm,
               stok_hbm, sw_hbm, inv0_hbm, inv1_hbm, te_hbm,
               e0v, e1v, w0v, w1v, offv, tmpv, stokv, swv, inv0v, inv1v, tev):
        wid = lax.axis_index("s") * 2 + lax.axis_index("c")

        @pl.when(wid == 0)
        def _():
            pltpu.sync_copy(e0_hbm, e0v)
            pltpu.sync_copy(e1_hbm, e1v)
            pltpu.sync_copy(w0_hbm, w0v)
            pltpu.sync_copy(w1_hbm, w1v)

            io = lax.iota(jnp.int32, L)
            zero16 = jnp.zeros((L,), jnp.int32)

            def place(keys, vals):
                ks, vs = plsc.sort_key_val(keys, vals)
                tmpv[...] = ks
                prev = plsc.load_gather(tmpv, [jnp.maximum(io - 1, 0)])
                nxt = plsc.load_gather(tmpv, [jnp.minimum(io + 1, L - 1)])
                is_new = (ks != prev) | (io == 0)
                first = plsc.cummax(jnp.where(is_new, io, 0))
                rank = io - first
                offk = plsc.load_gather(offv, [ks])
                dest = offk + rank
                is_last = (io == L - 1) | (ks != nxt)
                plsc.store_scatter(offv, [ks], dest + 1, mask=is_last)
                return vs, dest

            offv[...] = zero16

            def hist_body(j, _):
                toks = io + j * L
                place(e0v[pl.ds(j * L, L)], toks)
                place(e1v[pl.ds(j * L, L)], toks)
                return 0

            lax.fori_loop(0, t // L, hist_body, 0)

            c = offv[...]
            pc = ((c + (TM - 1)) >> TM_LOG2) << TM_LOG2
            po_incl = plsc.cumsum(pc)
            offv[...] = po_incl - pc
            cum_nt = po_incl >> TM_LOG2
            last_e = jnp.max(jnp.where(c > 0, io, 0))
            for ci in range(nt // L):
                tvec = io + ci * L
                acc = zero16
                for e in range(n_experts):
                    ce = jnp.max(jnp.where(io == e, cum_nt, 0))
                    acc = acc + (tvec >= ce).astype(jnp.int32)
                tev[pl.ds(ci * L, L)] = jnp.minimum(acc, last_e)

            def zinit(j, _):
                stokv[pl.ds(j * L, L)] = (io + j * L) & (t - 1)
                swv[pl.ds(j * L, L)] = jnp.zeros((L,), jnp.float32)
                return 0

            lax.fori_loop(0, ntot // L, zinit, 0)

            def place_body(j, _):
                toks = io + j * L
                vs0, d0 = place(e0v[pl.ds(j * L, L)], toks)
                plsc.store_scatter(stokv, [d0], vs0)
                plsc.store_scatter(swv, [d0], plsc.load_gather(w0v, [vs0]))
                plsc.store_scatter(inv0v, [vs0], d0)
                vs1, d1 = place(e1v[pl.ds(j * L, L)], toks)
                plsc.store_scatter(stokv, [d1], vs1)
                plsc.store_scatter(swv, [d1], plsc.load_gather(w1v, [vs1]))
                plsc.store_scatter(inv1v, [vs1], d1)
                return 0

            lax.fori_loop(0, t // L, place_body, 0)

            pltpu.sync_copy(stokv, stok_hbm)
            pltpu.sync_copy(swv, sw_hbm)
            pltpu.sync_copy(inv0v, inv0_hbm)
            pltpu.sync_copy(inv1v, inv1_hbm)
            pltpu.sync_copy(tev, te_hbm)

    return router


def _make_gather(t, d, ntot):
    mesh = plsc.VectorSubcoreMesh(core_axis_name="c", subcore_axis_name="s")
    nw = 32
    rows_per_w = ntot // nw
    cr = 16
    chunks = rows_per_w // cr
    nbuf = 4 if chunks % 4 == 0 else 2

    @functools.partial(
        pl.kernel,
        out_type=jax.ShapeDtypeStruct((ntot, d), jnp.int32),
        mesh=mesh,
        scratch_types=(
            [pltpu.VMEM((rows_per_w,), jnp.int32)]
            + [pltpu.VMEM((cr, d), jnp.int32) for _ in range(nbuf)]
            + [pltpu.SemaphoreType.DMA for _ in range(2 * nbuf)]
        ),
    )
    def gather(x_hbm, stok_hbm, xs_hbm, idxv, *bufsem):
        bufs = bufsem[:nbuf]
        gsem = bufsem[nbuf:2 * nbuf]
        wsem = bufsem[2 * nbuf:]
        wid = lax.axis_index("s") * 2 + lax.axis_index("c")
        base = wid * rows_per_w
        pltpu.sync_copy(stok_hbm.at[pl.ds(base, rows_per_w)], idxv)

        def clamp(ci, _):
            sl = pl.ds(ci * L, L)
            idxv[sl] = jnp.clip(idxv[sl], 0, t - 1)
            return 0

        lax.fori_loop(0, rows_per_w // L, clamp, 0)

        def gth(c, buf, sem):
            return pltpu.make_async_copy(
                x_hbm.at[idxv.at[pl.ds(c * cr, cr)]], buf, sem)

        def wrt(c, buf, sem):
            return pltpu.make_async_copy(
                buf, xs_hbm.at[pl.ds(base + c * cr, cr)], sem)

        for b in range(nbuf):
            gth(b, bufs[b], gsem[b]).start()

        def body(i, _):
            for b in range(nbuf):
                c = i * nbuf + b
                gth(c, bufs[b], gsem[b]).wait()
                wrt(c, bufs[b], wsem[b]).start()
                nc = c + nbuf

                @pl.when(nc < chunks)
                def _():
                    wrt(c, bufs[b], wsem[b]).wait()
                    gth(nc, bufs[b], gsem[b]).start()
            return 0

        lax.fori_loop(0, chunks // nbuf, body, 0)
        for b in range(nbuf):
            wrt(chunks - nbuf + b, bufs[b], wsem[b]).wait()

    return gather


def _gmm_compute(xs_ref, sw_ref, wg_ref, wu_ref, wd_ref, ys_ref):
    xi = xs_ref[...]
    x = pltpu.bitcast(xi, jnp.bfloat16)
    x = x.reshape(xi.shape[0], xi.shape[1] * 2)
    g = jnp.dot(x, wg_ref[0].astype(jnp.bfloat16),
                preferred_element_type=jnp.float32)
    u = jnp.dot(x, wu_ref[0].astype(jnp.bfloat16),
                preferred_element_type=jnp.float32)
    h = (g * jax.nn.sigmoid(g)) * u * sw_ref[0]
    y = jnp.dot(h.astype(jnp.bfloat16), wd_ref[0].astype(jnp.bfloat16),
                preferred_element_type=jnp.float32)
    dw = y.shape[1] // 2
    yi = lax.bitcast_convert_type(y, jnp.int32)
    lo = yi[:, :dw]
    hi = yi[:, dw:]
    lob = ((lo + 0x7FFF + ((lo >> 16) & 1)) >> 16) & 0xFFFF
    hib = ((hi + 0x7FFF + ((hi >> 16) & 1)) >> 16) << 16
    ys_ref[...] = hib | lob


def _gmm_body(te_ref, xs_ref, sw_ref, wg_ref, wu_ref, wd_ref, prev_ref,
              ys_ref):
    del prev_ref
    _gmm_compute(xs_ref, sw_ref, wg_ref, wu_ref, wd_ref, ys_ref)


def _gmm_body_fresh(te_ref, xs_ref, sw_ref, wg_ref, wu_ref, wd_ref, ys_ref):
    _gmm_compute(xs_ref, sw_ref, wg_ref, wu_ref, wd_ref, ys_ref)


def _gmm(te, xs, swr, w_gate, w_up, w_down, ys_prev, tile0, ngrid, ntot):
    _, d, f = w_gate.shape
    in_specs = [
        pl.BlockSpec((TM, d // 2), lambda i, te_r: (i, 0)),
        pl.BlockSpec((1, TM, 1), lambda i, te_r: (i, 0, 0)),
        pl.BlockSpec((1, d, f), lambda i, te_r: (te_r[i], 0, 0)),
        pl.BlockSpec((1, d, f), lambda i, te_r: (te_r[i], 0, 0)),
        pl.BlockSpec((1, f, d), lambda i, te_r: (te_r[i], 0, 0)),
    ]
    args = [te, xs, swr, w_gate, w_up, w_down]
    aliases = {}
    body = _gmm_body_fresh
    if ys_prev is not None:
        in_specs.append(pl.BlockSpec((TM, d // 2), lambda i, te_r: (0, 0)))
        args.append(ys_prev)
        aliases = {6: 0}
        body = _gmm_body
    grid_spec = pltpu.PrefetchScalarGridSpec(
        num_scalar_prefetch=1,
        grid=(ngrid,),
        in_specs=in_specs,
        out_specs=pl.BlockSpec((TM, d // 2), lambda i, te_r: (i + tile0, 0)),
    )
    return pl.pallas_call(
        body,
        grid_spec=grid_spec,
        out_shape=jax.ShapeDtypeStruct((ntot, d // 2), jnp.int32),
        input_output_aliases=aliases,
    )(*args)


def _make_combine(t, d, ntot):
    mesh = plsc.VectorSubcoreMesh(core_axis_name="c", subcore_axis_name="s")
    nw = 32
    rows_per_w = t // nw
    chunks = rows_per_w // L
    d_vecs = d // L

    cr = 8
    pairs = rows_per_w // (2 * cr)
    dw = d // 2

    @functools.partial(
        pl.kernel,
        out_type=jax.ShapeDtypeStruct((t, d), jnp.float32),
        mesh=mesh,
        compiler_params=pltpu.CompilerParams(needs_layout_passes=False),
        scratch_types=[
            pltpu.VMEM((rows_per_w,), jnp.int32),
            pltpu.VMEM((rows_per_w,), jnp.int32),
            pltpu.VMEM((cr, dw), jnp.int32),
            pltpu.VMEM((cr, dw), jnp.int32),
            pltpu.VMEM((cr, dw), jnp.int32),
            pltpu.VMEM((cr, dw), jnp.int32),
            pltpu.VMEM((cr, d), jnp.float32),
            pltpu.VMEM((cr, d), jnp.float32),
            pltpu.SemaphoreType.DMA,
            pltpu.SemaphoreType.DMA,
            pltpu.SemaphoreType.DMA,
            pltpu.SemaphoreType.DMA,
            pltpu.SemaphoreType.DMA,
            pltpu.SemaphoreType.DMA,
        ],
    )
    def combine(ys_hbm, inv0_hbm, inv1_hbm, y_hbm,
                idx0v, idx1v, a0, b0, a1, b1, o0, o1,
                ga0, gb0, ga1, gb1, wa0, wa1):
        wid = lax.axis_index("s") * 2 + lax.axis_index("c")
        base = wid * rows_per_w
        pltpu.sync_copy(inv0_hbm.at[pl.ds(base, rows_per_w)], idx0v)
        pltpu.sync_copy(inv1_hbm.at[pl.ds(base, rows_per_w)], idx1v)

        def clamp(ci, _):
            sl = pl.ds(ci * L, L)
            idx0v[sl] = jnp.clip(idx0v[sl], 0, ntot - 1)
            idx1v[sl] = jnp.clip(idx1v[sl], 0, ntot - 1)
            return 0

        lax.fori_loop(0, rows_per_w // L, clamp, 0)

        def gth(c, idxv, buf, sem):
            return pltpu.make_async_copy(
                ys_hbm.at[idxv.at[pl.ds(c * cr, cr)]], buf, sem)

        def wrt(c, buf, sem):
            return pltpu.make_async_copy(
                buf, y_hbm.at[pl.ds(base + c * cr, cr)], sem)

        himask = jnp.full((L,), -65536, jnp.int32)

        def addrows(av, bv, ov):
            def row(r, _):
                def col(j, _):
                    for u in range(4):
                        s = j * (4 * L) + u * L
                        aw = av[r, pl.ds(s, L)]
                        bw = bv[r, pl.ds(s, L)]
                        sm = plsc.bitcast(
                            plsc.bitcast(aw, jnp.bfloat16)
                            + plsc.bitcast(bw, jnp.bfloat16), jnp.int32)
                        ov[r, pl.ds(s, L)] = plsc.bitcast(
                            sm << 16, jnp.float32)
                        ov[r, pl.ds(dw + s, L)] = plsc.bitcast(
                            sm & himask, jnp.float32)
                    return 0

                lax.fori_loop(0, dw // (4 * L), col, 0)
                return 0

            lax.fori_loop(0, cr, row, 0)

        def body(i, _):
            c0 = 2 * i
            c1 = 2 * i + 1

            @pl.when(i > 0)
            def _():
                wrt(c0 - 2, o0, wa0).wait()

            gth(c0, idx0v, a0, ga0).start()
            gth(c0, idx1v, b0, gb0).start()

            @pl.when(i > 0)
            def _():
                wrt(c1 - 2, o1, wa1).wait()

            gth(c1, idx0v, a1, ga1).start()
            gth(c1, idx1v, b1, gb1).start()

            gth(c0, idx0v, a0, ga0).wait()
            gth(c0, idx1v, b0, gb0).wait()
            addrows(a0, b0, o0)
            wrt(c0, o0, wa0).start()

            gth(c1, idx0v, a1, ga1).wait()
            gth(c1, idx1v, b1, gb1).wait()
            addrows(a1, b1, o1)
            wrt(c1, o1, wa1).start()
            return 0

        lax.fori_loop(0, pairs, body, 0)
        wrt(2 * pairs - 2, o0, wa0).wait()
        wrt(2 * pairs - 1, o1, wa1).wait()

    return combine


def kernel(hidden_states, gate_weight, w_gate, w_up, w_down):
    bsz, seq, d = hidden_states.shape
    n_experts, _, f = w_gate.shape
    x = hidden_states.reshape(-1, d)
    t = x.shape[0]
    k = 2
    nt = (t * k) // TM + n_experts
    ntot = nt * TM

    e0, e1, w0, w1, xb = _gating(x, gate_weight)
    stok, sw, inv0, inv1, te = _make_router(t, n_experts, nt, ntot)(
        e0, e1, w0, w1)
    swr = sw.reshape(nt, TM, 1)
    xs = _make_gather(t, d // 2, ntot)(xb, stok)
    ys = _gmm(te, xs, swr, w_gate, w_up, w_down, None, 0, nt, ntot)
    y = _make_combine(t, d, ntot)(ys, inv0, inv1)
    return y.reshape(bsz, seq, d)


# speedup vs baseline: 1.7025x; 1.0046x over previous
import functools

import jax
import jax.numpy as jnp
from jax import lax
from jax.experimental import pallas as pl
from jax.experimental.pallas import tpu as pltpu
from jax.experimental.pallas import tpu_sc as plsc

TM = 128
TM_LOG2 = 7
L = 16


def _gate_body(x_ref, gw_ref, e0_ref, e1_ref, w0_ref, w1_ref, xb_ref, *,
               n_experts):
    x = x_ref[...]
    t = x.shape[0]
    xbf = x.astype(jnp.bfloat16).reshape(2 * t, x.shape[1] // 2)
    xb_ref[...] = pltpu.bitcast(xbf, jnp.int32)
    logits = lax.dot_general(x, gw_ref[...], (((1,), (1,)), ((), ())),
                             preferred_element_type=jnp.float32)
    m = jnp.max(logits, axis=-1, keepdims=True)
    p = jnp.exp(logits - m)
    p = p / jnp.sum(p, axis=-1, keepdims=True)
    eio = lax.broadcasted_iota(jnp.int32, p.shape, 1)
    m1 = jnp.max(p, axis=-1, keepdims=True)
    i1 = jnp.min(jnp.where(p == m1, eio, n_experts), axis=-1, keepdims=True)
    p2 = jnp.where(eio == i1, -1.0, p)
    m2 = jnp.max(p2, axis=-1, keepdims=True)
    i2 = jnp.min(jnp.where(p2 == m2, eio, n_experts), axis=-1, keepdims=True)
    denom = m1 + m2 + 1e-20
    e0_ref[...] = i1
    e1_ref[...] = i2
    w0_ref[...] = m1 / denom
    w1_ref[...] = m2 / denom


def _gating(x, gate_weight):
    t, _ = x.shape
    n_experts = gate_weight.shape[0]
    outs = pl.pallas_call(
        functools.partial(_gate_body, n_experts=n_experts),
        out_shape=[
            jax.ShapeDtypeStruct((t, 1), jnp.int32),
            jax.ShapeDtypeStruct((t, 1), jnp.int32),
            jax.ShapeDtypeStruct((t, 1), jnp.float32),
            jax.ShapeDtypeStruct((t, 1), jnp.float32),
            jax.ShapeDtypeStruct((t, gate_weight.shape[1] // 2), jnp.int32),
        ],
    )(x, gate_weight)
    e0, e1, w0, w1, xb = outs
    return (e0.reshape(t), e1.reshape(t), w0.reshape(t), w1.reshape(t), xb)


def _make_route_gather(t, n_experts, nt, ntot, d):
    mesh = plsc.VectorSubcoreMesh(core_axis_name="c", subcore_axis_name="s")
    nw = 32
    rows_per_w = ntot // nw
    cr = 16
    chunks = rows_per_w // cr
    nbuf = 4 if chunks % 4 == 0 else 2

    @functools.partial(
        pl.kernel,
        out_type=[
            jax.ShapeDtypeStruct((ntot, d), jnp.int32),
            jax.ShapeDtypeStruct((ntot,), jnp.float32),
            jax.ShapeDtypeStruct((t,), jnp.int32),
            jax.ShapeDtypeStruct((t,), jnp.int32),
            jax.ShapeDtypeStruct((nt,), jnp.int32),
            jax.ShapeDtypeStruct((2 * ntot,), jnp.int32),
        ],
        mesh=mesh,
        compiler_params=pltpu.CompilerParams(needs_layout_passes=False),
        scratch_types=(
            [
                pltpu.VMEM((t,), jnp.int32),
                pltpu.VMEM((t,), jnp.int32),
                pltpu.VMEM((t,), jnp.float32),
                pltpu.VMEM((t,), jnp.float32),
                pltpu.VMEM((L,), jnp.int32),
                pltpu.VMEM((L,), jnp.int32),
                pltpu.VMEM((ntot,), jnp.int32),
                pltpu.VMEM((ntot,), jnp.float32),
                pltpu.VMEM((t,), jnp.int32),
                pltpu.VMEM((t,), jnp.int32),
                pltpu.VMEM((nt,), jnp.int32),
                pltpu.VMEM((rows_per_w,), jnp.int32),
            ]
            + [pltpu.VMEM((cr, d), jnp.int32) for _ in range(nbuf)]
            + [pltpu.SemaphoreType.DMA for _ in range(2 * nbuf)]
        ),
    )
    def router(e0_hbm, e1_hbm, w0_hbm, w1_hbm, x_hbm,
               xs_hbm, sw_hbm, inv0_hbm, inv1_hbm, te_hbm, stok2_hbm,
               e0v, e1v, w0v, w1v, offv, tmpv, stokv, swv, inv0v, inv1v, tev,
               idxv, *bufsem):
        bufs = bufsem[:nbuf]
        gsem = bufsem[nbuf:2 * nbuf]
        wsem = bufsem[2 * nbuf:]
        cid = lax.axis_index("c")
        sid = lax.axis_index("s")

        @pl.when(sid == 0)
        def _():
            pltpu.sync_copy(e0_hbm, e0v)
            pltpu.sync_copy(e1_hbm, e1v)
            pltpu.sync_copy(w0_hbm, w0v)
            pltpu.sync_copy(w1_hbm, w1v)

            io = lax.iota(jnp.int32, L)
            zero16 = jnp.zeros((L,), jnp.int32)

            def place(keys, vals):
                ks, vs = plsc.sort_key_val(keys, vals)
                tmpv[...] = ks
                prev = plsc.load_gather(tmpv, [jnp.maximum(io - 1, 0)])
                nxt = plsc.load_gather(tmpv, [jnp.minimum(io + 1, L - 1)])
                is_new = (ks != prev) | (io == 0)
                first = plsc.cummax(jnp.where(is_new, io, 0))
                rank = io - first
                offk = plsc.load_gather(offv, [ks])
                dest = offk + rank
                is_last = (io == L - 1) | (ks != nxt)
                plsc.store_scatter(offv, [ks], dest + 1, mask=is_last)
                return vs, dest

            offv[...] = zero16

            def hist_body(j, _):
                toks = io + j * L
                place(e0v[pl.ds(j * L, L)], toks)
                place(e1v[pl.ds(j * L, L)], toks)
                return 0

            lax.fori_loop(0, t // L, hist_body, 0)

            c = offv[...]
            pc = ((c + (TM - 1)) >> TM_LOG2) << TM_LOG2
            po_incl = plsc.cumsum(pc)
            offv[...] = po_incl - pc
            cum_nt = po_incl >> TM_LOG2
            last_e = jnp.max(jnp.where(c > 0, io, 0))
            for ci in range(nt // L):
                tvec = io + ci * L
                acc = zero16
                for e in range(n_experts):
                    ce = jnp.max(jnp.where(io == e, cum_nt, 0))
                    acc = acc + (tvec >= ce).astype(jnp.int32)
                tev[pl.ds(ci * L, L)] = jnp.minimum(acc, last_e)

            def zinit(j, _):
                stokv[pl.ds(j * L, L)] = (io + j * L) & (t - 1)
                swv[pl.ds(j * L, L)] = jnp.zeros((L,), jnp.float32)
                return 0

            lax.fori_loop(0, ntot // L, zinit, 0)

            def place_body(j, _):
                toks = io + j * L
                vs0, d0 = place(e0v[pl.ds(j * L, L)], toks)
                plsc.store_scatter(stokv, [d0], vs0)
                plsc.store_scatter(swv, [d0], plsc.load_gather(w0v, [vs0]))
                plsc.store_scatter(inv0v, [vs0], d0)
                vs1, d1 = place(e1v[pl.ds(j * L, L)], toks)
                plsc.store_scatter(stokv, [d1], vs1)
                plsc.store_scatter(swv, [d1], plsc.load_gather(w1v, [vs1]))
                plsc.store_scatter(inv1v, [vs1], d1)
                return 0

            lax.fori_loop(0, t // L, place_body, 0)

            pltpu.sync_copy(stokv, stok2_hbm.at[pl.ds(cid * ntot, ntot)])

            @pl.when(cid == 0)
            def _():
                pltpu.sync_copy(swv, sw_hbm)
                pltpu.sync_copy(inv0v, inv0_hbm)
                pltpu.sync_copy(inv1v, inv1_hbm)
                pltpu.sync_copy(tev, te_hbm)

        plsc.subcore_barrier()

        wid = sid * 2 + cid
        base = wid * rows_per_w
        pltpu.sync_copy(
            stok2_hbm.at[pl.ds(cid * ntot + base, rows_per_w)], idxv)

        def clamp(ci, _):
            sl = pl.ds(ci * L, L)
            idxv[sl] = jnp.clip(idxv[sl], 0, t - 1)
            return 0

        lax.fori_loop(0, rows_per_w // L, clamp, 0)

        def gth(c, buf, sem):
            return pltpu.make_async_copy(
                x_hbm.at[idxv.at[pl.ds(c * cr, cr)]], buf, sem)

        def wrt(c, buf, sem):
            return pltpu.make_async_copy(
                buf, xs_hbm.at[pl.ds(base + c * cr, cr)], sem)

        for b in range(nbuf):
            gth(b, bufs[b], gsem[b]).start()

        def body(i, _):
            for b in range(nbuf):
                c = i * nbuf + b
                gth(c, bufs[b], gsem[b]).wait()
                wrt(c, bufs[b], wsem[b]).start()
                nc = c + nbuf

                @pl.when(nc < chunks)
                def _():
                    wrt(c, bufs[b], wsem[b]).wait()
                    gth(nc, bufs[b], gsem[b]).start()
            return 0

        lax.fori_loop(0, chunks // nbuf, body, 0)
        for b in range(nbuf):
            wrt(chunks - nbuf + b, bufs[b], wsem[b]).wait()

    return router


def _gmm_compute(xs_ref, sw_ref, wg_ref, wu_ref, wd_ref, ys_ref):
    xi = xs_ref[...]
    x = pltpu.bitcast(xi, jnp.bfloat16)
    x = x.reshape(xi.shape[0], xi.shape[1] * 2)
    g = jnp.dot(x, wg_ref[0].astype(jnp.bfloat16),
                preferred_element_type=jnp.float32)
    u = jnp.dot(x, wu_ref[0].astype(jnp.bfloat16),
                preferred_element_type=jnp.float32)
    h = (g * jax.nn.sigmoid(g)) * u * sw_ref[0]
    y = jnp.dot(h.astype(jnp.bfloat16), wd_ref[0].astype(jnp.bfloat16),
                preferred_element_type=jnp.float32)
    dw = y.shape[1] // 2
    yi = lax.bitcast_convert_type(y, jnp.int32)
    lo = yi[:, :dw]
    hi = yi[:, dw:]
    lob = ((lo + 0x7FFF + ((lo >> 16) & 1)) >> 16) & 0xFFFF
    hib = ((hi + 0x7FFF + ((hi >> 16) & 1)) >> 16) << 16
    ys_ref[...] = hib | lob


def _gmm_body(te_ref, xs_ref, sw_ref, wg_ref, wu_ref, wd_ref, prev_ref,
              ys_ref):
    del prev_ref
    _gmm_compute(xs_ref, sw_ref, wg_ref, wu_ref, wd_ref, ys_ref)


def _gmm_body_fresh(te_ref, xs_ref, sw_ref, wg_ref, wu_ref, wd_ref, ys_ref):
    _gmm_compute(xs_ref, sw_ref, wg_ref, wu_ref, wd_ref, ys_ref)


def _gmm(te, xs, swr, w_gate, w_up, w_down, ys_prev, tile0, ngrid, ntot):
    _, d, f = w_gate.shape
    in_specs = [
        pl.BlockSpec((TM, d // 2), lambda i, te_r: (i, 0)),
        pl.BlockSpec((1, TM, 1), lambda i, te_r: (i, 0, 0)),
        pl.BlockSpec((1, d, f), lambda i, te_r: (te_r[i], 0, 0)),
        pl.BlockSpec((1, d, f), lambda i, te_r: (te_r[i], 0, 0)),
        pl.BlockSpec((1, f, d), lambda i, te_r: (te_r[i], 0, 0)),
    ]
    args = [te, xs, swr, w_gate, w_up, w_down]
    aliases = {}
    body = _gmm_body_fresh
    if ys_prev is not None:
        in_specs.append(pl.BlockSpec((TM, d // 2), lambda i, te_r: (0, 0)))
        args.append(ys_prev)
        aliases = {6: 0}
        body = _gmm_body
    grid_spec = pltpu.PrefetchScalarGridSpec(
        num_scalar_prefetch=1,
        grid=(ngrid,),
        in_specs=in_specs,
        out_specs=pl.BlockSpec((TM, d // 2), lambda i, te_r: (i + tile0, 0)),
    )
    return pl.pallas_call(
        body,
        grid_spec=grid_spec,
        out_shape=jax.ShapeDtypeStruct((ntot, d // 2), jnp.int32),
        input_output_aliases=aliases,
    )(*args)


def _make_combine(t, d, ntot):
    mesh = plsc.VectorSubcoreMesh(core_axis_name="c", subcore_axis_name="s")
    nw = 32
    rows_per_w = t // nw
    chunks = rows_per_w // L
    d_vecs = d // L

    cr = 8
    pairs = rows_per_w // (2 * cr)
    dw = d // 2

    @functools.partial(
        pl.kernel,
        out_type=jax.ShapeDtypeStruct((t, d), jnp.float32),
        mesh=mesh,
        compiler_params=pltpu.CompilerParams(needs_layout_passes=False),
        scratch_types=[
            pltpu.VMEM((rows_per_w,), jnp.int32),
            pltpu.VMEM((rows_per_w,), jnp.int32),
            pltpu.VMEM((cr, dw), jnp.int32),
            pltpu.VMEM((cr, dw), jnp.int32),
            pltpu.VMEM((cr, dw), jnp.int32),
            pltpu.VMEM((cr, dw), jnp.int32),
            pltpu.VMEM((cr, d), jnp.float32),
            pltpu.VMEM((cr, d), jnp.float32),
            pltpu.SemaphoreType.DMA,
            pltpu.SemaphoreType.DMA,
            pltpu.SemaphoreType.DMA,
            pltpu.SemaphoreType.DMA,
            pltpu.SemaphoreType.DMA,
            pltpu.SemaphoreType.DMA,
        ],
    )
    def combine(ys_hbm, inv0_hbm, inv1_hbm, y_hbm,
                idx0v, idx1v, a0, b0, a1, b1, o0, o1,
                ga0, gb0, ga1, gb1, wa0, wa1):
        wid = lax.axis_index("s") * 2 + lax.axis_index("c")
        base = wid * rows_per_w
        pltpu.sync_copy(inv0_hbm.at[pl.ds(base, rows_per_w)], idx0v)
        pltpu.sync_copy(inv1_hbm.at[pl.ds(base, rows_per_w)], idx1v)

        def clamp(ci, _):
            sl = pl.ds(ci * L, L)
            idx0v[sl] = jnp.clip(idx0v[sl], 0, ntot - 1)
            idx1v[sl] = jnp.clip(idx1v[sl], 0, ntot - 1)
            return 0

        lax.fori_loop(0, rows_per_w // L, clamp, 0)

        def gth(c, idxv, buf, sem):
            return pltpu.make_async_copy(
                ys_hbm.at[idxv.at[pl.ds(c * cr, cr)]], buf, sem)

        def wrt(c, buf, sem):
            return pltpu.make_async_copy(
                buf, y_hbm.at[pl.ds(base + c * cr, cr)], sem)

        himask = jnp.full((L,), -65536, jnp.int32)

        def addrows(av, bv, ov):
            def row(r, _):
                def col(j, _):
                    for u in range(4):
                        s = j * (4 * L) + u * L
                        aw = av[r, pl.ds(s, L)]
                        bw = bv[r, pl.ds(s, L)]
                        sm = plsc.bitcast(
                            plsc.bitcast(aw, jnp.bfloat16)
                            + plsc.bitcast(bw, jnp.bfloat16), jnp.int32)
                        ov[r, pl.ds(s, L)] = plsc.bitcast(
                            sm << 16, jnp.float32)
                        ov[r, pl.ds(dw + s, L)] = plsc.bitcast(
                            sm & himask, jnp.float32)
                    return 0

                lax.fori_loop(0, dw // (4 * L), col, 0)
                return 0

            lax.fori_loop(0, cr, row, 0)

        def body(i, _):
            c0 = 2 * i
            c1 = 2 * i + 1

            @pl.when(i > 0)
            def _():
                wrt(c0 - 2, o0, wa0).wait()

            gth(c0, idx0v, a0, ga0).start()
            gth(c0, idx1v, b0, gb0).start()

            @pl.when(i > 0)
            def _():
                wrt(c1 - 2, o1, wa1).wait()

            gth(c1, idx0v, a1, ga1).start()
            gth(c1, idx1v, b1, gb1).start()

            gth(c0, idx0v, a0, ga0).wait()
            gth(c0, idx1v, b0, gb0).wait()
            addrows(a0, b0, o0)
            wrt(c0, o0, wa0).start()

            gth(c1, idx0v, a1, ga1).wait()
            gth(c1, idx1v, b1, gb1).wait()
            addrows(a1, b1, o1)
            wrt(c1, o1, wa1).start()
            return 0

        lax.fori_loop(0, pairs, body, 0)
        wrt(2 * pairs - 2, o0, wa0).wait()
        wrt(2 * pairs - 1, o1, wa1).wait()

    return combine


def kernel(hidden_states, gate_weight, w_gate, w_up, w_down):
    bsz, seq, d = hidden_states.shape
    n_experts, _, f = w_gate.shape
    x = hidden_states.reshape(-1, d)
    t = x.shape[0]
    k = 2
    nt = (t * k) // TM + n_experts
    ntot = nt * TM

    e0, e1, w0, w1, xb = _gating(x, gate_weight)
    xs, sw, inv0, inv1, te, _ = _make_route_gather(
        t, n_experts, nt, ntot, d // 2)(e0, e1, w0, w1, xb)
    swr = sw.reshape(nt, TM, 1)
    ys = _gmm(te, xs, swr, w_gate, w_up, w_down, None, 0, nt, ntot)
    y = _make_combine(t, d, ntot)(ys, inv0, inv1)
    return y.reshape(bsz, seq, d)
